# Initial kernel scaffold; baseline (speedup 1.0000x reference)
import jax
import jax.numpy as jnp
from jax.experimental import pallas as pl


def kernel(x_table, x_column, edge_index_tc, edge_index_ct, batch_table, batch_column, queries, W_src_tc, W_dst_tc, a_src_tc, a_dst_tc, W_src_ct, W_dst_ct, a_src_ct, a_dst_ct, W_out_t, b_out_t, W_out_c, b_out_c, W_t1, b_t1, W_t2, b_t2, W_c1, b_c1, W_c2, b_c2):
    raise NotImplementedError("write your pallas kernel here")


# speedup vs baseline: 6.1446x; 6.1446x over previous
import functools

import jax
import jax.numpy as jnp
from jax import lax
from jax.experimental import pallas as pl
from jax.experimental.pallas import tpu as pltpu
from jax.experimental.pallas import tpu_sc as plsc

F32 = jnp.float32
I32 = jnp.int32

D = 128
SW = 32
NSLAB = D // SW
NCORE = 2
NSUB = 16
NLANE = 16
EB = 128


def _pre_body(x_ref, ws_ref, as_ref, wd_ref, ad_ref, hs_ref, ss_ref, sd_ref):
    x = x_ref[...]
    hs = jnp.dot(x, ws_ref[...], preferred_element_type=F32)
    hs_ref[...] = hs
    ss_ref[...] = jnp.dot(hs, as_ref[...], preferred_element_type=F32)
    hd = jnp.dot(x, wd_ref[...], preferred_element_type=F32)
    sd_ref[...] = jnp.dot(hd, ad_ref[...], preferred_element_type=F32)


def _dense_pre(x, w_src, a_src, w_dst, a_dst, bs):
    n = x.shape[0]
    return pl.pallas_call(
        _pre_body,
        grid=(n // bs,),
        in_specs=[
            pl.BlockSpec((bs, D), lambda i: (i, 0)),
            pl.BlockSpec((D, D), lambda i: (0, 0)),
            pl.BlockSpec((D, 1), lambda i: (0, 0)),
            pl.BlockSpec((D, D), lambda i: (0, 0)),
            pl.BlockSpec((D, 1), lambda i: (0, 0)),
        ],
        out_specs=[
            pl.BlockSpec((bs, D), lambda i: (i, 0)),
            pl.BlockSpec((bs, 1), lambda i: (i, 0)),
            pl.BlockSpec((bs, 1), lambda i: (i, 0)),
        ],
        out_shape=[
            jax.ShapeDtypeStruct((n, D), F32),
            jax.ShapeDtypeStruct((n, 1), F32),
            jax.ShapeDtypeStruct((n, 1), F32),
        ],
    )(x, w_src, a_src.reshape(D, 1), w_dst, a_dst.reshape(D, 1))


def _qw_body(q_ref, wt_ref, wc_ref, qt_ref, qc_ref):
    q = q_ref[...]
    qt_ref[...] = jnp.dot(q, wt_ref[...], preferred_element_type=F32)
    qc_ref[...] = jnp.dot(q, wc_ref[...], preferred_element_type=F32)


def _dense_qw(queries, w_t1q, w_c1q):
    b = queries.shape[0]
    return pl.pallas_call(
        _qw_body,
        grid=(1,),
        in_specs=[
            pl.BlockSpec((b, D), lambda i: (0, 0)),
            pl.BlockSpec((D, D), lambda i: (0, 0)),
            pl.BlockSpec((D, D), lambda i: (0, 0)),
        ],
        out_specs=[
            pl.BlockSpec((b, D), lambda i: (0, 0)),
            pl.BlockSpec((b, D), lambda i: (0, 0)),
        ],
        out_shape=[
            jax.ShapeDtypeStruct((b, D), F32),
            jax.ShapeDtypeStruct((b, D), F32),
        ],
    )(queries, w_t1q, w_c1q)


def _sc_all(nt, nc, epad, rc, rt, nst, nsc, b):
    ept = epad // (NSUB * NCORE)
    nb = ept // EB
    ngrp = EB // NLANE

    mesh = plsc.VectorSubcoreMesh(
        core_axis_name="c", subcore_axis_name="s",
        num_cores=NCORE, num_subcores=NSUB)

    @functools.partial(
        pl.kernel,
        out_type=(
            jax.ShapeDtypeStruct((NCORE, NSLAB + 1, rc, SW), F32),
            jax.ShapeDtypeStruct((NCORE, NSLAB + 1, rt, SW), F32),
            jax.ShapeDtypeStruct((NSLAB, rc, SW), F32),
            jax.ShapeDtypeStruct((NSLAB, rt, SW), F32),
            jax.ShapeDtypeStruct((epad,), F32),
            jax.ShapeDtypeStruct((epad,), F32),
        ),
        mesh=mesh,
        compiler_params=pltpu.CompilerParams(
            needs_layout_passes=False, use_tc_tiling_on_sc=False),
        scratch_types=[
            pltpu.VMEM((EB, SW), F32),
            pltpu.VMEM((EB, SW), F32),
            pltpu.VMEM((EB, SW), F32),
            pltpu.VMEM((64, SW), F32),
            pltpu.VMEM((EB,), I32),
            pltpu.VMEM((EB,), I32),
            pltpu.VMEM((EB,), I32),
            pltpu.VMEM((EB,), F32),
            pltpu.VMEM((EB,), I32),
            pltpu.VMEM((EB,), I32),
            pltpu.VMEM((EB,), I32),
            pltpu.VMEM_SHARED((nst, SW), F32),
            pltpu.VMEM_SHARED((nsc, SW), F32),
            pltpu.VMEM_SHARED((nsc, SW), F32),
            pltpu.VMEM_SHARED((nst, SW), F32),
            pltpu.VMEM_SHARED((rc, SW), F32),
            pltpu.SemaphoreType.DMA,
        ],
    )
    def sc_fn(hs_t_ref, hs_c_ref, ss_tc_ref, sd_tc_ref, ss_ct_ref, sd_ct_ref,
              src_tc_ref, dst_tc_ref, src_ct_ref, dst_ct_ref,
              qw_c_ref, qidx_c_ref, qw_t_ref, qidx_t_ref,
              agg_c_ref, agg_t_ref, qg_c_ref, qg_t_ref, p_tc_ref, p_ct_ref,
              rows_v, rss, rsd, zbuf, srcb, dstb, dloc, p_b,
              ridx_a, ridx_b, qidxb,
              ss_t_sh, sd_c_sh, ss_c_sh, sd_t_sh, agg_sh, sem):
        c = lax.axis_index("c")
        s = lax.axis_index("s")
        wid = c * NSUB + s
        iot = jnp.arange(NLANE, dtype=I32)

        @pl.when(s == 0)
        def _stage():
            pltpu.sync_copy(ss_tc_ref, ss_t_sh)
            pltpu.sync_copy(sd_tc_ref, sd_c_sh)
            pltpu.sync_copy(ss_ct_ref, ss_c_sh)
            pltpu.sync_copy(sd_ct_ref, sd_t_sh)

        @pl.loop(0, 64)
        def _zb(i):
            for j in range(SW // NLANE):
                zbuf[i, pl.ds(j * NLANE, NLANE)] = jnp.zeros((NLANE,), F32)

        def q_gather(qidx_ref, qw_ref, qg_ref, nqb, nq):
            @pl.loop(wid, nqb, step=NCORE * NSUB)
            def _qg(blk):
                pltpu.sync_copy(qidx_ref.at[pl.ds(blk * EB, EB)], qidxb)
                for k in range(NSLAB):
                    @pl.loop(0, ngrp)
                    def _adj(jj):
                        off = jj * NLANE
                        qidx = qidxb[pl.ds(off, NLANE)]
                        ridx_a[pl.ds(off, NLANE)] = qidx + k * nq
                    pltpu.async_copy(qw_ref.at[ridx_a], rows_v, sem).wait()
                    pltpu.sync_copy(rows_v,
                                    qg_ref.at[k, pl.ds(blk * EB, EB)])

        q_gather(qidx_c_ref, qw_c_ref, qg_c_ref, rc // EB, b)
        q_gather(qidx_t_ref, qw_t_ref, qg_t_ref, rt // EB, b)

        plsc.subcore_barrier()

        def clear_acc(span):
            nfull = span // 64
            rem = span % 64
            @pl.loop(0, nfull)
            def _zz(i):
                pltpu.sync_copy(zbuf, agg_sh.at[pl.ds(s * span + i * 64, 64)])
            if rem:
                pltpu.sync_copy(zbuf.at[pl.ds(0, rem)],
                                agg_sh.at[pl.ds(s * span + nfull * 64, rem)])
            plsc.subcore_barrier()

        def direction(hs_ref, src_ref, dst_ref, ss_sh, sd_sh, p_ref,
                      agg_ref, n_src, n_dst, rd):
            ebase = wid * ept
            span = rd // NSUB
            stripe = rd // NSUB

            @pl.loop(0, EB)
            def _zr(i):
                for j in range(SW // NLANE):
                    rows_v[i, pl.ds(j * NLANE, NLANE)] = jnp.zeros((NLANE,), F32)
            clear_acc(span)

            @pl.loop(0, nb)
            def _zbatch(ib):
                eb = ebase + ib * EB
                pltpu.sync_copy(src_ref.at[pl.ds(eb, EB)], srcb)
                pltpu.sync_copy(dst_ref.at[pl.ds(eb, EB)], dstb)

                @pl.loop(0, ngrp)
                def _mkidx(jj):
                    off = jj * NLANE
                    si = srcb[pl.ds(off, NLANE)]
                    di = dstb[pl.ds(off, NLANE)]
                    dg = jnp.minimum(jnp.maximum(di, 0), n_dst - 1)
                    ridx_a[pl.ds(off, NLANE)] = si >> 5
                    ridx_b[pl.ds(off, NLANE)] = dg >> 5

                pltpu.async_copy(ss_sh.at[ridx_a], rss, sem).wait()
                pltpu.async_copy(sd_sh.at[ridx_b], rsd, sem).wait()

                @pl.loop(0, ngrp)
                def _score(jj):
                    off = jj * NLANE
                    ioff = iot + off
                    si = srcb[pl.ds(off, NLANE)]
                    di = dstb[pl.ds(off, NLANE)]
                    dg = jnp.minimum(jnp.maximum(di, 0), n_dst - 1)
                    sv = plsc.load_gather(rss, [ioff, si & 31])
                    dv = plsc.load_gather(rsd, [ioff, dg & 31])
                    al = sv + dv
                    al = jnp.where(al >= 0.0, al, al * 0.2)
                    p = jnp.exp(al)
                    p_b[pl.ds(off, NLANE)] = p
                    plsc.store_scatter(rows_v, [ioff, jnp.zeros((NLANE,), I32)], p)
                    ok = (di >= 0) & (di < n_dst)
                    dloc[pl.ds(off, NLANE)] = jnp.where(ok, di, n_dst)

                pltpu.sync_copy(p_b, p_ref.at[pl.ds(eb, EB)])
                pltpu.sync_copy(rows_v, agg_sh.at[dloc], add=True)

            plsc.subcore_barrier()
            pltpu.sync_copy(agg_sh.at[pl.ds(s * stripe, stripe)],
                            agg_ref.at[c, NSLAB, pl.ds(s * stripe, stripe)])
            plsc.subcore_barrier()

            for k in range(NSLAB):
                clear_acc(span)

                @pl.loop(0, nb)
                def _sbatch(ib):
                    eb = ebase + ib * EB
                    pltpu.sync_copy(src_ref.at[pl.ds(eb, EB)], srcb)
                    pltpu.sync_copy(dst_ref.at[pl.ds(eb, EB)], dstb)
                    pltpu.sync_copy(p_ref.at[pl.ds(eb, EB)], p_b)

                    @pl.loop(0, ngrp)
                    def _mkidx2(jj):
                        off = jj * NLANE
                        si = srcb[pl.ds(off, NLANE)]
                        di = dstb[pl.ds(off, NLANE)]
                        ridx_a[pl.ds(off, NLANE)] = si + k * n_src
                        ok = (di >= 0) & (di < n_dst)
                        dloc[pl.ds(off, NLANE)] = jnp.where(ok, di, n_dst)

                    pltpu.async_copy(hs_ref.at[ridx_a], rows_v, sem).wait()

                    @pl.loop(0, ngrp)
                    def _scale(g):
                        row0 = g * NLANE
                        for ii in range(NLANE):
                            pv = plsc.load_gather(
                                p_b, [jnp.full((NLANE,), row0 + ii, I32)])
                            r = row0 + ii
                            for j in range(SW // NLANE):
                                rows_v[r, pl.ds(j * NLANE, NLANE)] = (
                                    rows_v[r, pl.ds(j * NLANE, NLANE)] * pv)

                    pltpu.sync_copy(rows_v, agg_sh.at[dloc], add=True)

                plsc.subcore_barrier()
                pltpu.sync_copy(agg_sh.at[pl.ds(s * stripe, stripe)],
                                agg_ref.at[c, k, pl.ds(s * stripe, stripe)])
                plsc.subcore_barrier()

        direction(hs_t_ref, src_tc_ref, dst_tc_ref, ss_t_sh, sd_c_sh,
                  p_tc_ref, agg_c_ref, nt, nc, rc)
        direction(hs_c_ref, src_ct_ref, dst_ct_ref, ss_c_sh, sd_t_sh,
                  p_ct_ref, agg_t_ref, nc, nt, rt)

    return sc_fn


def _head_body(a0, a1, z0, z1, qg, wo, bo, w1, b1, w2, b2, out):
    agg = (a0[...] + a1[...]) / (z0[...] + z1[...] + 1e-16)
    f = jnp.dot(jnp.maximum(agg, 0.0), wo[...], preferred_element_type=F32) + bo[...]
    h = jnp.maximum(jnp.dot(f, w1[...], preferred_element_type=F32) + qg[...] + b1[...], 0.0)
    out[...] = jnp.dot(h, w2[...], preferred_element_type=F32) + b2[...]


def _head(a0, a1, z0, z1, qg, wo, bo, w1, b1, w2, b2, bs):
    n = qg.shape[0]
    row = lambda i: (i, 0)
    fix = lambda i: (0, 0)
    out = pl.pallas_call(
        _head_body,
        grid=(n // bs,),
        in_specs=[
            pl.BlockSpec((bs, D), row), pl.BlockSpec((bs, D), row),
            pl.BlockSpec((bs, 1), row), pl.BlockSpec((bs, 1), row),
            pl.BlockSpec((bs, D), row),
            pl.BlockSpec((D, D), fix), pl.BlockSpec((1, D), fix),
            pl.BlockSpec((D, D), fix), pl.BlockSpec((1, D), fix),
            pl.BlockSpec((D, 1), fix), pl.BlockSpec((1, 1), fix),
        ],
        out_specs=pl.BlockSpec((bs, 1), row),
        out_shape=jax.ShapeDtypeStruct((n, 1), F32),
    )(a0, a1, z0, z1, qg, wo, bo.reshape(1, D), w1, b1.reshape(1, D),
      w2, b2.reshape(1, 1))
    return out[:, 0]


def kernel(x_table, x_column, edge_index_tc, edge_index_ct, batch_table,
           batch_column, queries, W_src_tc, W_dst_tc, a_src_tc, a_dst_tc,
           W_src_ct, W_dst_ct, a_src_ct, a_dst_ct, W_out_t, b_out_t,
           W_out_c, b_out_c, W_t1, b_t1, W_t2, b_t2, W_c1, b_c1, W_c2, b_c2):
    nt = x_table.shape[0]
    nc = x_column.shape[0]
    e = edge_index_tc.shape[1]
    b = queries.shape[0]

    hs_tc, ss_tc, sd_ct = _dense_pre(x_table, W_src_tc, a_src_tc,
                                     W_dst_ct, a_dst_ct, bs=2000)
    hs_ct, ss_ct, sd_tc = _dense_pre(x_column, W_src_ct, a_src_ct,
                                     W_dst_tc, a_dst_tc, bs=2000)
    qw_t, qw_c = _dense_qw(queries, W_t1[D:], W_c1[D:])

    def _slab_stack(m):
        return jnp.concatenate([m[:, k * SW:(k + 1) * SW] for k in range(NSLAB)], 0)

    hs_t_all = _slab_stack(hs_tc)
    hs_c_all = _slab_stack(hs_ct)
    qw_t_all = _slab_stack(qw_t)
    qw_c_all = _slab_stack(qw_c)

    def _pack_scores(v, n):
        npad = ((n + SW - 1) // SW) * SW
        return jnp.concatenate([v[:, 0], jnp.zeros((npad - n,), F32)]).reshape(-1, SW)

    ss_tc_p = _pack_scores(ss_tc, nt)
    sd_tc_p = _pack_scores(sd_tc, nc)
    ss_ct_p = _pack_scores(ss_ct, nc)
    sd_ct_p = _pack_scores(sd_ct, nt)
    nst = ss_tc_p.shape[0]
    nsc = sd_tc_p.shape[0]

    quantum = NCORE * NSUB * EB
    epad = ((e + quantum - 1) // quantum) * quantum
    pad = epad - e

    def _pad_edges(ei):
        src = jnp.concatenate([ei[0].astype(I32), jnp.zeros((pad,), I32)])
        dst = jnp.concatenate([ei[1].astype(I32), jnp.full((pad,), -1, I32)])
        return src, dst

    src_tc, dst_tc = _pad_edges(edge_index_tc)
    src_ct, dst_ct = _pad_edges(edge_index_ct)

    rc = ((nc + EB) // EB) * EB
    rt = ((nt + EB) // EB) * EB
    qidx_c = jnp.concatenate(
        [batch_column.astype(I32), jnp.zeros((rc - nc,), I32)])
    qidx_t = jnp.concatenate(
        [batch_table.astype(I32), jnp.zeros((rt - nt,), I32)])

    sc_fn = _sc_all(nt, nc, epad, rc, rt, nst, nsc, b)
    agg_c5, agg_t5, qg_c4, qg_t4, _p1, _p2 = sc_fn(
        hs_t_all, hs_c_all, ss_tc_p, sd_tc_p, ss_ct_p, sd_ct_p,
        src_tc, dst_tc, src_ct, dst_ct,
        qw_c_all, qidx_c, qw_t_all, qidx_t)

    def _unslab(a4, n):
        return jnp.concatenate([a4[k, :n] for k in range(NSLAB)], axis=1)

    agg_c0 = _unslab(agg_c5[0, :NSLAB], nc)
    agg_c1 = _unslab(agg_c5[1, :NSLAB], nc)
    z_c0 = agg_c5[0, NSLAB, :nc, 0:1]
    z_c1 = agg_c5[1, NSLAB, :nc, 0:1]
    agg_t0 = _unslab(agg_t5[0, :NSLAB], nt)
    agg_t1 = _unslab(agg_t5[1, :NSLAB], nt)
    z_t0 = agg_t5[0, NSLAB, :nt, 0:1]
    z_t1 = agg_t5[1, NSLAB, :nt, 0:1]
    qg_c = _unslab(qg_c4, nc)
    qg_t = _unslab(qg_t4, nt)

    table_probs = _head(agg_t0, agg_t1, z_t0, z_t1, qg_t,
                        W_out_t, b_out_t, W_t1[:D], b_t1, W_t2, b_t2, bs=2000)
    column_probs = _head(agg_c0, agg_c1, z_c0, z_c1, qg_c,
                         W_out_c, b_out_c, W_c1[:D], b_c1, W_c2, b_c2, bs=2000)

    return (table_probs, column_probs)


# speedup vs baseline: 7.8477x; 1.2772x over previous
import functools

import jax
import jax.numpy as jnp
from jax import lax
from jax.experimental import pallas as pl
from jax.experimental.pallas import tpu as pltpu
from jax.experimental.pallas import tpu_sc as plsc

F32 = jnp.float32
I32 = jnp.int32

D = 128
SW = 32
NSLAB = D // SW
NCORE = 2
NSUB = 16
NLANE = 16
EB = 128


def _pre_body(x_ref, ws_ref, as_ref, wd_ref, ad_ref, hs_ref, ss_ref, sd_ref):
    x = x_ref[...]
    hs = jnp.dot(x, ws_ref[...], preferred_element_type=F32)
    hs_ref[...] = hs
    ss_ref[...] = jnp.dot(hs, as_ref[...], preferred_element_type=F32)
    hd = jnp.dot(x, wd_ref[...], preferred_element_type=F32)
    sd_ref[...] = jnp.dot(hd, ad_ref[...], preferred_element_type=F32)


def _dense_pre(x, w_src, a_src, w_dst, a_dst, bs):
    n = x.shape[0]
    return pl.pallas_call(
        _pre_body,
        grid=(n // bs,),
        in_specs=[
            pl.BlockSpec((bs, D), lambda i: (i, 0)),
            pl.BlockSpec((D, D), lambda i: (0, 0)),
            pl.BlockSpec((D, 1), lambda i: (0, 0)),
            pl.BlockSpec((D, D), lambda i: (0, 0)),
            pl.BlockSpec((D, 1), lambda i: (0, 0)),
        ],
        out_specs=[
            pl.BlockSpec((bs, D), lambda i: (i, 0)),
            pl.BlockSpec((bs, 1), lambda i: (i, 0)),
            pl.BlockSpec((bs, 1), lambda i: (i, 0)),
        ],
        out_shape=[
            jax.ShapeDtypeStruct((n, D), F32),
            jax.ShapeDtypeStruct((n, 1), F32),
            jax.ShapeDtypeStruct((n, 1), F32),
        ],
    )(x, w_src, a_src.reshape(D, 1), w_dst, a_dst.reshape(D, 1))


def _qw_body(q_ref, wt_ref, wc_ref, qt_ref, qc_ref):
    q = q_ref[...]
    qt_ref[...] = jnp.dot(q, wt_ref[...], preferred_element_type=F32)
    qc_ref[...] = jnp.dot(q, wc_ref[...], preferred_element_type=F32)


def _dense_qw(queries, w_t1q, w_c1q):
    b = queries.shape[0]
    return pl.pallas_call(
        _qw_body,
        grid=(1,),
        in_specs=[
            pl.BlockSpec((b, D), lambda i: (0, 0)),
            pl.BlockSpec((D, D), lambda i: (0, 0)),
            pl.BlockSpec((D, D), lambda i: (0, 0)),
        ],
        out_specs=[
            pl.BlockSpec((b, D), lambda i: (0, 0)),
            pl.BlockSpec((b, D), lambda i: (0, 0)),
        ],
        out_shape=[
            jax.ShapeDtypeStruct((b, D), F32),
            jax.ShapeDtypeStruct((b, D), F32),
        ],
    )(queries, w_t1q, w_c1q)


def _sc_all(nt, nc, epad, rc, rt, nst, nsc, b):
    ept = epad // (NSUB * NCORE)
    DB = 2 * EB
    nb2 = ept // DB
    ngrp = EB // NLANE
    sh_tc = (nt - 1).bit_length()
    sh_ct = (nc - 1).bit_length()

    mesh = plsc.VectorSubcoreMesh(
        core_axis_name="c", subcore_axis_name="s",
        num_cores=NCORE, num_subcores=NSUB)

    @functools.partial(
        pl.kernel,
        out_type=(
            jax.ShapeDtypeStruct((NCORE, NSLAB + 1, rc, SW), F32),
            jax.ShapeDtypeStruct((NCORE, NSLAB + 1, rt, SW), F32),
            jax.ShapeDtypeStruct((NSLAB, rc, SW), F32),
            jax.ShapeDtypeStruct((NSLAB, rt, SW), F32),
            jax.ShapeDtypeStruct((epad,), F32),
            jax.ShapeDtypeStruct((epad,), F32),
        ),
        mesh=mesh,
        compiler_params=pltpu.CompilerParams(
            needs_layout_passes=False, use_tc_tiling_on_sc=False),
        scratch_types=[
            pltpu.VMEM((EB, SW), F32),
            pltpu.VMEM((EB, SW), F32),
            pltpu.VMEM((EB, SW), F32),
            pltpu.VMEM((EB, SW), F32),
            pltpu.VMEM((64, SW), F32),
            pltpu.VMEM((DB,), I32),
            pltpu.VMEM((DB,), F32),
            pltpu.VMEM((EB,), I32),
            pltpu.VMEM((EB,), I32),
            pltpu.VMEM((EB,), I32),
            pltpu.VMEM((EB,), I32),
            pltpu.VMEM((EB,), I32),
            pltpu.VMEM_SHARED((nst, SW), F32),
            pltpu.VMEM_SHARED((nsc, SW), F32),
            pltpu.VMEM_SHARED((nsc, SW), F32),
            pltpu.VMEM_SHARED((nst, SW), F32),
            pltpu.VMEM_SHARED((rc, SW), F32),
            pltpu.SemaphoreType.DMA,
        ],
    )
    def sc_fn(hs_t_ref, hs_c_ref, ss_tc_ref, sd_tc_ref, ss_ct_ref, sd_ct_ref,
              code_tc_ref, code_ct_ref,
              qw_c_ref, qidx_c_ref, qw_t_ref, qidx_t_ref,
              agg_c_ref, agg_t_ref, qg_c_ref, qg_t_ref, p_tc_ref, p_ct_ref,
              rows_a, rows_b, rss, rsd, zbuf, codeb, p_b,
              ridx_a, ridx_b, dloc0, dloc1, qidxb,
              ss_t_sh, sd_c_sh, ss_c_sh, sd_t_sh, agg_sh, sem):
        c = lax.axis_index("c")
        s = lax.axis_index("s")
        wid = c * NSUB + s
        iot = jnp.arange(NLANE, dtype=I32)

        @pl.when(s == 0)
        def _stage():
            pltpu.sync_copy(ss_tc_ref, ss_t_sh)
            pltpu.sync_copy(sd_tc_ref, sd_c_sh)
            pltpu.sync_copy(ss_ct_ref, ss_c_sh)
            pltpu.sync_copy(sd_ct_ref, sd_t_sh)

        @pl.loop(0, 64)
        def _zb(i):
            for j in range(SW // NLANE):
                zbuf[i, pl.ds(j * NLANE, NLANE)] = jnp.zeros((NLANE,), F32)

        def q_gather(qidx_ref, qw_ref, qg_ref, nqb, nq):
            @pl.loop(wid, nqb, step=NCORE * NSUB)
            def _qg(blk):
                pltpu.sync_copy(qidx_ref.at[pl.ds(blk * EB, EB)], qidxb)
                for k in range(NSLAB):
                    @pl.loop(0, ngrp)
                    def _adj(jj):
                        off = jj * NLANE
                        qidx = qidxb[pl.ds(off, NLANE)]
                        ridx_a[pl.ds(off, NLANE)] = qidx + k * nq
                    pltpu.async_copy(qw_ref.at[ridx_a], rows_a, sem).wait()
                    pltpu.sync_copy(rows_a,
                                    qg_ref.at[k, pl.ds(blk * EB, EB)])

        q_gather(qidx_c_ref, qw_c_ref, qg_c_ref, rc // EB, b)
        q_gather(qidx_t_ref, qw_t_ref, qg_t_ref, rt // EB, b)

        plsc.subcore_barrier()

        def clear_acc(span):
            nfull = span // 64
            rem = span % 64
            @pl.loop(0, nfull)
            def _zz(i):
                pltpu.sync_copy(zbuf, agg_sh.at[pl.ds(s * span + i * 64, 64)])
            if rem:
                pltpu.sync_copy(zbuf.at[pl.ds(0, rem)],
                                agg_sh.at[pl.ds(s * span + nfull * 64, rem)])
            plsc.subcore_barrier()

        def direction(hs_ref, code_ref, ss_sh, sd_sh, p_ref,
                      agg_ref, n_src, n_dst, rd, shift):
            ebase = wid * ept
            stripe = rd // NSUB
            mask = (1 << shift) - 1
            dlocs = (dloc0, dloc1)

            @pl.loop(0, EB)
            def _zr(i):
                for j in range(SW // NLANE):
                    rows_a[i, pl.ds(j * NLANE, NLANE)] = jnp.zeros((NLANE,), F32)
                    rows_b[i, pl.ds(j * NLANE, NLANE)] = jnp.zeros((NLANE,), F32)
            clear_acc(stripe)

            @pl.loop(0, nb2)
            def _zbatch(ib):
                eb = ebase + ib * DB
                pltpu.sync_copy(code_ref.at[pl.ds(eb, DB)], codeb)

                for h in range(2):
                    ho = h * EB

                    @pl.loop(0, ngrp)
                    def _mkidx(jj):
                        off = jj * NLANE
                        cd = codeb[pl.ds(ho + off, NLANE)]
                        si = cd & mask
                        di = cd >> shift
                        dg = jnp.minimum(di, n_dst - 1)
                        ridx_a[pl.ds(off, NLANE)] = si >> 5
                        ridx_b[pl.ds(off, NLANE)] = dg >> 5
                        dlocs[h][pl.ds(off, NLANE)] = di

                    ga = pltpu.async_copy(ss_sh.at[ridx_a], rss, sem)
                    gb = pltpu.async_copy(sd_sh.at[ridx_b], rsd, sem)
                    ga.wait()
                    gb.wait()

                    @pl.loop(0, ngrp)
                    def _score(jj):
                        off = jj * NLANE
                        ioff = iot + off
                        cd = codeb[pl.ds(ho + off, NLANE)]
                        si = cd & mask
                        dg = jnp.minimum(cd >> shift, n_dst - 1)
                        sv = plsc.load_gather(rss, [ioff, si & 31])
                        dv = plsc.load_gather(rsd, [ioff, dg & 31])
                        al = sv + dv
                        al = jnp.where(al >= 0.0, al, al * 0.2)
                        p = jnp.exp(al)
                        p_b[pl.ds(ho + off, NLANE)] = p
                        plsc.store_scatter(
                            (rows_a, rows_b)[h],
                            [ioff, jnp.zeros((NLANE,), I32)], p)

                pltpu.sync_copy(p_b, p_ref.at[pl.ds(eb, DB)])
                pltpu.sync_copy(rows_a, agg_sh.at[dloc0], add=True)
                pltpu.sync_copy(rows_b, agg_sh.at[dloc1], add=True)

            plsc.subcore_barrier()
            pltpu.sync_copy(agg_sh.at[pl.ds(s * stripe, stripe)],
                            agg_ref.at[c, NSLAB, pl.ds(s * stripe, stripe)])
            plsc.subcore_barrier()

            for k in range(NSLAB):
                clear_acc(stripe)

                @pl.loop(0, nb2)
                def _sbatch(ib):
                    eb = ebase + ib * DB
                    ca = pltpu.async_copy(code_ref.at[pl.ds(eb, DB)], codeb, sem)
                    cb = pltpu.async_copy(p_ref.at[pl.ds(eb, DB)], p_b, sem)
                    ca.wait()
                    cb.wait()

                    @pl.loop(0, ngrp)
                    def _mkidx2(jj):
                        off = jj * NLANE
                        cd0 = codeb[pl.ds(off, NLANE)]
                        cd1 = codeb[pl.ds(EB + off, NLANE)]
                        ridx_a[pl.ds(off, NLANE)] = (cd0 & mask) + k * n_src
                        ridx_b[pl.ds(off, NLANE)] = (cd1 & mask) + k * n_src
                        dloc0[pl.ds(off, NLANE)] = cd0 >> shift
                        dloc1[pl.ds(off, NLANE)] = cd1 >> shift

                    ga = pltpu.async_copy(hs_ref.at[ridx_a], rows_a, sem)
                    gb = pltpu.async_copy(hs_ref.at[ridx_b], rows_b, sem)
                    ga.wait()
                    gb.wait()

                    for h, rows_h in ((0, rows_a), (1, rows_b)):
                        @pl.loop(0, ngrp)
                        def _scale(g):
                            row0 = g * NLANE
                            for ii in range(NLANE):
                                pv = plsc.load_gather(
                                    p_b, [jnp.full((NLANE,),
                                                   h * EB + row0 + ii, I32)])
                                r = row0 + ii
                                for j in range(SW // NLANE):
                                    rows_h[r, pl.ds(j * NLANE, NLANE)] = (
                                        rows_h[r, pl.ds(j * NLANE, NLANE)] * pv)

                    pltpu.sync_copy(rows_a, agg_sh.at[dloc0], add=True)
                    pltpu.sync_copy(rows_b, agg_sh.at[dloc1], add=True)

                plsc.subcore_barrier()
                pltpu.sync_copy(agg_sh.at[pl.ds(s * stripe, stripe)],
                                agg_ref.at[c, k, pl.ds(s * stripe, stripe)])
                plsc.subcore_barrier()

        direction(hs_t_ref, code_tc_ref, ss_t_sh, sd_c_sh,
                  p_tc_ref, agg_c_ref, nt, nc, rc, sh_tc)
        direction(hs_c_ref, code_ct_ref, ss_c_sh, sd_t_sh,
                  p_ct_ref, agg_t_ref, nc, nt, rt, sh_ct)

    return sc_fn


def _head_body(a0, a1, z0, z1, qg, wo, bo, w1, b1, w2, b2, out):
    agg = (a0[...] + a1[...]) / (z0[...] + z1[...] + 1e-16)
    f = jnp.dot(jnp.maximum(agg, 0.0), wo[...], preferred_element_type=F32) + bo[...]
    h = jnp.maximum(jnp.dot(f, w1[...], preferred_element_type=F32) + qg[...] + b1[...], 0.0)
    out[...] = jnp.dot(h, w2[...], preferred_element_type=F32) + b2[...]


def _head(a0, a1, z0, z1, qg, wo, bo, w1, b1, w2, b2, bs):
    n = qg.shape[0]
    row = lambda i: (i, 0)
    fix = lambda i: (0, 0)
    out = pl.pallas_call(
        _head_body,
        grid=(n // bs,),
        in_specs=[
            pl.BlockSpec((bs, D), row), pl.BlockSpec((bs, D), row),
            pl.BlockSpec((bs, 1), row), pl.BlockSpec((bs, 1), row),
            pl.BlockSpec((bs, D), row),
            pl.BlockSpec((D, D), fix), pl.BlockSpec((1, D), fix),
            pl.BlockSpec((D, D), fix), pl.BlockSpec((1, D), fix),
            pl.BlockSpec((D, 1), fix), pl.BlockSpec((1, 1), fix),
        ],
        out_specs=pl.BlockSpec((bs, 1), row),
        out_shape=jax.ShapeDtypeStruct((n, 1), F32),
    )(a0, a1, z0, z1, qg, wo, bo.reshape(1, D), w1, b1.reshape(1, D),
      w2, b2.reshape(1, 1))
    return out[:, 0]


def kernel(x_table, x_column, edge_index_tc, edge_index_ct, batch_table,
           batch_column, queries, W_src_tc, W_dst_tc, a_src_tc, a_dst_tc,
           W_src_ct, W_dst_ct, a_src_ct, a_dst_ct, W_out_t, b_out_t,
           W_out_c, b_out_c, W_t1, b_t1, W_t2, b_t2, W_c1, b_c1, W_c2, b_c2):
    nt = x_table.shape[0]
    nc = x_column.shape[0]
    e = edge_index_tc.shape[1]
    b = queries.shape[0]

    hs_tc, ss_tc, sd_ct = _dense_pre(x_table, W_src_tc, a_src_tc,
                                     W_dst_ct, a_dst_ct, bs=2000)
    hs_ct, ss_ct, sd_tc = _dense_pre(x_column, W_src_ct, a_src_ct,
                                     W_dst_tc, a_dst_tc, bs=2000)
    qw_t, qw_c = _dense_qw(queries, W_t1[D:], W_c1[D:])

    def _slab_stack(m):
        return jnp.concatenate([m[:, k * SW:(k + 1) * SW] for k in range(NSLAB)], 0)

    hs_t_all = _slab_stack(hs_tc)
    hs_c_all = _slab_stack(hs_ct)
    qw_t_all = _slab_stack(qw_t)
    qw_c_all = _slab_stack(qw_c)

    def _pack_scores(v, n):
        npad = ((n + SW - 1) // SW) * SW
        return jnp.concatenate([v[:, 0], jnp.zeros((npad - n,), F32)]).reshape(-1, SW)

    ss_tc_p = _pack_scores(ss_tc, nt)
    sd_tc_p = _pack_scores(sd_tc, nc)
    ss_ct_p = _pack_scores(ss_ct, nc)
    sd_ct_p = _pack_scores(sd_ct, nt)
    nst = ss_tc_p.shape[0]
    nsc = sd_tc_p.shape[0]

    quantum = NCORE * NSUB * 2 * EB
    epad = ((e + quantum - 1) // quantum) * quantum
    pad = epad - e

    def _pack_edges(ei, n_src, n_dst):
        shift = (n_src - 1).bit_length()
        src = jnp.concatenate([ei[0].astype(I32), jnp.zeros((pad,), I32)])
        dst = jnp.concatenate([ei[1].astype(I32),
                               jnp.full((pad,), n_dst, I32)])
        return src | (dst << shift)

    code_tc = _pack_edges(edge_index_tc, nt, nc)
    code_ct = _pack_edges(edge_index_ct, nc, nt)

    rc = ((nc + EB) // EB) * EB
    rt = ((nt + EB) // EB) * EB
    qidx_c = jnp.concatenate(
        [batch_column.astype(I32), jnp.zeros((rc - nc,), I32)])
    qidx_t = jnp.concatenate(
        [batch_table.astype(I32), jnp.zeros((rt - nt,), I32)])

    sc_fn = _sc_all(nt, nc, epad, rc, rt, nst, nsc, b)
    agg_c5, agg_t5, qg_c4, qg_t4, _p1, _p2 = sc_fn(
        hs_t_all, hs_c_all, ss_tc_p, sd_tc_p, ss_ct_p, sd_ct_p,
        code_tc, code_ct,
        qw_c_all, qidx_c, qw_t_all, qidx_t)

    def _unslab(a4, n):
        return jnp.concatenate([a4[k, :n] for k in range(NSLAB)], axis=1)

    agg_c0 = _unslab(agg_c5[0, :NSLAB], nc)
    agg_c1 = _unslab(agg_c5[1, :NSLAB], nc)
    z_c0 = agg_c5[0, NSLAB, :nc, 0:1]
    z_c1 = agg_c5[1, NSLAB, :nc, 0:1]
    agg_t0 = _unslab(agg_t5[0, :NSLAB], nt)
    agg_t1 = _unslab(agg_t5[1, :NSLAB], nt)
    z_t0 = agg_t5[0, NSLAB, :nt, 0:1]
    z_t1 = agg_t5[1, NSLAB, :nt, 0:1]
    qg_c = _unslab(qg_c4, nc)
    qg_t = _unslab(qg_t4, nt)

    table_probs = _head(agg_t0, agg_t1, z_t0, z_t1, qg_t,
                        W_out_t, b_out_t, W_t1[:D], b_t1, W_t2, b_t2, bs=2000)
    column_probs = _head(agg_c0, agg_c1, z_c0, z_c1, qg_c,
                         W_out_c, b_out_c, W_c1[:D], b_c1, W_c2, b_c2, bs=2000)

    return (table_probs, column_probs)


# speedup vs baseline: 8.2071x; 1.0458x over previous
import functools

import jax
import jax.numpy as jnp
from jax import lax
from jax.experimental import pallas as pl
from jax.experimental.pallas import tpu as pltpu
from jax.experimental.pallas import tpu_sc as plsc

F32 = jnp.float32
I32 = jnp.int32

D = 128
SW = 32
NSLAB = D // SW
NCORE = 2
NSUB = 16
NLANE = 16
EB = 128


def _pre_body(x_ref, ws_ref, as_ref, wd_ref, ad_ref, hs_ref, ss_ref, sd_ref):
    x = x_ref[...]
    hs = jnp.dot(x, ws_ref[...], preferred_element_type=F32)
    hs_ref[...] = hs
    ss_ref[...] = jnp.dot(hs, as_ref[...], preferred_element_type=F32)
    hd = jnp.dot(x, wd_ref[...], preferred_element_type=F32)
    sd_ref[...] = jnp.dot(hd, ad_ref[...], preferred_element_type=F32)


def _dense_pre(x, w_src, a_src, w_dst, a_dst, bs):
    n = x.shape[0]
    return pl.pallas_call(
        _pre_body,
        grid=(n // bs,),
        in_specs=[
            pl.BlockSpec((bs, D), lambda i: (i, 0)),
            pl.BlockSpec((D, D), lambda i: (0, 0)),
            pl.BlockSpec((D, 1), lambda i: (0, 0)),
            pl.BlockSpec((D, D), lambda i: (0, 0)),
            pl.BlockSpec((D, 1), lambda i: (0, 0)),
        ],
        out_specs=[
            pl.BlockSpec((bs, D), lambda i: (i, 0)),
            pl.BlockSpec((bs, 1), lambda i: (i, 0)),
            pl.BlockSpec((bs, 1), lambda i: (i, 0)),
        ],
        out_shape=[
            jax.ShapeDtypeStruct((n, D), F32),
            jax.ShapeDtypeStruct((n, 1), F32),
            jax.ShapeDtypeStruct((n, 1), F32),
        ],
    )(x, w_src, a_src.reshape(D, 1), w_dst, a_dst.reshape(D, 1))


def _qw_body(q_ref, wt_ref, wc_ref, qt_ref, qc_ref):
    q = q_ref[...]
    qt_ref[...] = jnp.dot(q, wt_ref[...], preferred_element_type=F32)
    qc_ref[...] = jnp.dot(q, wc_ref[...], preferred_element_type=F32)


def _dense_qw(queries, w_t1q, w_c1q):
    b = queries.shape[0]
    return pl.pallas_call(
        _qw_body,
        grid=(1,),
        in_specs=[
            pl.BlockSpec((b, D), lambda i: (0, 0)),
            pl.BlockSpec((D, D), lambda i: (0, 0)),
            pl.BlockSpec((D, D), lambda i: (0, 0)),
        ],
        out_specs=[
            pl.BlockSpec((b, D), lambda i: (0, 0)),
            pl.BlockSpec((b, D), lambda i: (0, 0)),
        ],
        out_shape=[
            jax.ShapeDtypeStruct((b, D), F32),
            jax.ShapeDtypeStruct((b, D), F32),
        ],
    )(queries, w_t1q, w_c1q)


def _sc_all(nt, nc, epad, rc, rt, nst, nsc, b):
    ept = epad // (NSUB * NCORE)
    DB = 2 * EB
    nb2 = ept // DB
    ngrp = EB // NLANE
    sh_tc = (nt - 1).bit_length()
    sh_ct = (nc - 1).bit_length()

    mesh = plsc.VectorSubcoreMesh(
        core_axis_name="c", subcore_axis_name="s",
        num_cores=NCORE, num_subcores=NSUB)

    @functools.partial(
        pl.kernel,
        out_type=(
            jax.ShapeDtypeStruct((NCORE, NSLAB + 1, rc, SW), F32),
            jax.ShapeDtypeStruct((NCORE, NSLAB + 1, rt, SW), F32),
            jax.ShapeDtypeStruct((NSLAB, rc, SW), F32),
            jax.ShapeDtypeStruct((NSLAB, rt, SW), F32),
            jax.ShapeDtypeStruct((epad,), F32),
            jax.ShapeDtypeStruct((epad,), F32),
        ),
        mesh=mesh,
        compiler_params=pltpu.CompilerParams(
            needs_layout_passes=False, use_tc_tiling_on_sc=False),
        scratch_types=[
            pltpu.VMEM((EB, SW), F32),
            pltpu.VMEM((EB, SW), F32),
            pltpu.VMEM((EB, SW), F32),
            pltpu.VMEM((EB, SW), F32),
            pltpu.VMEM((64, SW), F32),
            pltpu.VMEM((DB,), I32),
            pltpu.VMEM((DB,), F32),
            pltpu.VMEM((DB,), I32),
            pltpu.VMEM((DB,), F32),
            pltpu.VMEM((EB,), I32),
            pltpu.VMEM((EB,), I32),
            pltpu.VMEM((EB,), I32),
            pltpu.VMEM((EB,), I32),
            pltpu.VMEM((EB,), I32),
            pltpu.VMEM((EB,), I32),
            pltpu.VMEM((EB,), I32),
            pltpu.VMEM((EB,), I32),
            pltpu.VMEM((EB,), I32),
            pltpu.VMEM_SHARED((nst, SW), F32),
            pltpu.VMEM_SHARED((nsc, SW), F32),
            pltpu.VMEM_SHARED((nsc, SW), F32),
            pltpu.VMEM_SHARED((nst, SW), F32),
            pltpu.VMEM_SHARED((rc, SW), F32),
            pltpu.SemaphoreType.DMA,
            pltpu.SemaphoreType.DMA,
        ],
    )
    def sc_fn(hs_t_ref, hs_c_ref, ss_tc_ref, sd_tc_ref, ss_ct_ref, sd_ct_ref,
              code_tc_ref, code_ct_ref,
              qw_c_ref, qidx_c_ref, qw_t_ref, qidx_t_ref,
              agg_c_ref, agg_t_ref, qg_c_ref, qg_t_ref, p_tc_ref, p_ct_ref,
              rows_a, rows_b, rss, rsd, zbuf, codeb, p_b, codeb2, p_b2,
              ridx_a, ridx_b, ridx_a2, ridx_b2,
              dloc0, dloc1, dloc2, dloc3, qidxb,
              ss_t_sh, sd_c_sh, ss_c_sh, sd_t_sh, agg_sh, sem, sem_l):
        c = lax.axis_index("c")
        s = lax.axis_index("s")
        wid = c * NSUB + s
        iot = jnp.arange(NLANE, dtype=I32)

        @pl.when(s == 0)
        def _stage():
            pltpu.sync_copy(ss_tc_ref, ss_t_sh)
            pltpu.sync_copy(sd_tc_ref, sd_c_sh)
            pltpu.sync_copy(ss_ct_ref, ss_c_sh)
            pltpu.sync_copy(sd_ct_ref, sd_t_sh)

        @pl.loop(0, 64)
        def _zb(i):
            for j in range(SW // NLANE):
                zbuf[i, pl.ds(j * NLANE, NLANE)] = jnp.zeros((NLANE,), F32)

        def q_gather(qidx_ref, qw_ref, qg_ref, nqb, nq):
            @pl.loop(wid, nqb, step=NCORE * NSUB)
            def _qg(blk):
                pltpu.sync_copy(qidx_ref.at[pl.ds(blk * EB, EB)], qidxb)
                for k in range(NSLAB):
                    @pl.loop(0, ngrp)
                    def _adj(jj):
                        off = jj * NLANE
                        qidx = qidxb[pl.ds(off, NLANE)]
                        ridx_a[pl.ds(off, NLANE)] = qidx + k * nq
                    pltpu.async_copy(qw_ref.at[ridx_a], rows_a, sem).wait()
                    pltpu.sync_copy(rows_a,
                                    qg_ref.at[k, pl.ds(blk * EB, EB)])

        q_gather(qidx_c_ref, qw_c_ref, qg_c_ref, rc // EB, b)
        q_gather(qidx_t_ref, qw_t_ref, qg_t_ref, rt // EB, b)

        plsc.subcore_barrier()

        def clear_acc(span):
            nfull = span // 64
            rem = span % 64
            @pl.loop(0, nfull)
            def _zz(i):
                pltpu.sync_copy(zbuf, agg_sh.at[pl.ds(s * span + i * 64, 64)])
            if rem:
                pltpu.sync_copy(zbuf.at[pl.ds(0, rem)],
                                agg_sh.at[pl.ds(s * span + nfull * 64, rem)])
            plsc.subcore_barrier()

        def direction(hs_ref, code_ref, ss_sh, sd_sh, p_ref,
                      agg_ref, n_src, n_dst, rd, shift):
            ebase = wid * ept
            stripe = rd // NSUB
            mask = (1 << shift) - 1
            dlocs = (dloc0, dloc1)

            @pl.loop(0, EB)
            def _zr(i):
                for j in range(SW // NLANE):
                    rows_a[i, pl.ds(j * NLANE, NLANE)] = jnp.zeros((NLANE,), F32)
                    rows_b[i, pl.ds(j * NLANE, NLANE)] = jnp.zeros((NLANE,), F32)
            clear_acc(stripe)

            @pl.loop(0, nb2)
            def _zbatch(ib):
                eb = ebase + ib * DB
                pltpu.sync_copy(code_ref.at[pl.ds(eb, DB)], codeb)

                for h in range(2):
                    ho = h * EB

                    @pl.loop(0, ngrp)
                    def _mkidx(jj):
                        off = jj * NLANE
                        cd = codeb[pl.ds(ho + off, NLANE)]
                        si = cd & mask
                        di = cd >> shift
                        dg = jnp.minimum(di, n_dst - 1)
                        ridx_a[pl.ds(off, NLANE)] = si >> 5
                        ridx_b[pl.ds(off, NLANE)] = dg >> 5
                        dlocs[h][pl.ds(off, NLANE)] = di

                    ga = pltpu.async_copy(ss_sh.at[ridx_a], rss, sem)
                    gb = pltpu.async_copy(sd_sh.at[ridx_b], rsd, sem)
                    ga.wait()
                    gb.wait()

                    @pl.loop(0, ngrp)
                    def _score(jj):
                        off = jj * NLANE
                        ioff = iot + off
                        cd = codeb[pl.ds(ho + off, NLANE)]
                        si = cd & mask
                        dg = jnp.minimum(cd >> shift, n_dst - 1)
                        sv = plsc.load_gather(rss, [ioff, si & 31])
                        dv = plsc.load_gather(rsd, [ioff, dg & 31])
                        al = sv + dv
                        al = jnp.where(al >= 0.0, al, al * 0.2)
                        p = jnp.exp(al)
                        p_b[pl.ds(ho + off, NLANE)] = p
                        plsc.store_scatter(
                            (rows_a, rows_b)[h],
                            [ioff, jnp.zeros((NLANE,), I32)], p)

                pltpu.sync_copy(p_b, p_ref.at[pl.ds(eb, DB)])
                pltpu.sync_copy(rows_a, agg_sh.at[dloc0], add=True)
                pltpu.sync_copy(rows_b, agg_sh.at[dloc1], add=True)

            plsc.subcore_barrier()
            pltpu.sync_copy(agg_sh.at[pl.ds(s * stripe, stripe)],
                            agg_ref.at[c, NSLAB, pl.ds(s * stripe, stripe)])
            plsc.subcore_barrier()

            B0 = (codeb, p_b, ridx_a, ridx_b, dloc0, dloc1, rows_a, rows_b)
            B1 = (codeb2, p_b2, ridx_a2, ridx_b2, dloc2, dloc3, rss, rsd)

            @pl.loop(0, NSLAB)
            def _slab(k):
                clear_acc(stripe)

                def load_issue(ib, S):
                    eb = ebase + ib * DB
                    pltpu.async_copy(code_ref.at[pl.ds(eb, DB)], S[0], sem_l)
                    pltpu.async_copy(p_ref.at[pl.ds(eb, DB)], S[1], sem_l)

                def load_wait(S):
                    pltpu.make_async_copy(
                        code_ref.at[pl.ds(ebase, DB)], S[0], sem_l).wait()
                    pltpu.make_async_copy(
                        p_ref.at[pl.ds(ebase, DB)], S[1], sem_l).wait()

                def mkidx(S):
                    cb, _, ra, rb, d0, d1, _, _ = S

                    @pl.loop(0, ngrp)
                    def _mk(jj):
                        off = jj * NLANE
                        cd0 = cb[pl.ds(off, NLANE)]
                        cd1 = cb[pl.ds(EB + off, NLANE)]
                        ra[pl.ds(off, NLANE)] = (
                            jnp.minimum(cd0 & mask, n_src - 1) + k * n_src)
                        rb[pl.ds(off, NLANE)] = (
                            jnp.minimum(cd1 & mask, n_src - 1) + k * n_src)
                        d0[pl.ds(off, NLANE)] = jnp.minimum(cd0 >> shift, n_dst)
                        d1[pl.ds(off, NLANE)] = jnp.minimum(cd1 >> shift, n_dst)

                def gather_issue(S):
                    pltpu.async_copy(hs_ref.at[S[2]], S[6], sem)
                    pltpu.async_copy(hs_ref.at[S[3]], S[7], sem)

                def gather_wait(S):
                    pltpu.make_async_copy(hs_ref.at[S[2]], S[6], sem).wait()
                    pltpu.make_async_copy(hs_ref.at[S[3]], S[7], sem).wait()

                def scale_scatter(S):
                    _, pb, _, _, d0, d1, ra_, rb_ = S
                    for h, rows_h in ((0, ra_), (1, rb_)):
                        @pl.loop(0, ngrp)
                        def _scale(g):
                            row0 = g * NLANE
                            for ii in range(NLANE):
                                pv = plsc.load_gather(
                                    pb, [jnp.full((NLANE,),
                                                  h * EB + row0 + ii, I32)])
                                r = row0 + ii
                                for j in range(SW // NLANE):
                                    rows_h[r, pl.ds(j * NLANE, NLANE)] = (
                                        rows_h[r, pl.ds(j * NLANE, NLANE)] * pv)
                    pltpu.sync_copy(ra_, agg_sh.at[d0], add=True)
                    pltpu.sync_copy(rb_, agg_sh.at[d1], add=True)

                pltpu.sync_copy(code_ref.at[pl.ds(ebase, DB)], codeb)
                pltpu.sync_copy(p_ref.at[pl.ds(ebase, DB)], p_b)
                mkidx(B0)
                gather_issue(B0)

                npairs = (nb2 - 1) // 2

                @pl.loop(0, npairs)
                def _pair(i2):
                    for par, (cur, nxt) in ((0, (B0, B1)), (1, (B1, B0))):
                        ib = i2 * 2 + par
                        load_issue(ib + 1, nxt)
                        gather_wait(cur)
                        scale_scatter(cur)
                        load_wait(nxt)
                        mkidx(nxt)
                        gather_issue(nxt)

                for ib in range(2 * npairs, nb2):
                    cur, nxt = (B0, B1) if ib % 2 == 0 else (B1, B0)
                    if ib + 1 < nb2:
                        load_issue(ib + 1, nxt)
                    gather_wait(cur)
                    scale_scatter(cur)
                    if ib + 1 < nb2:
                        load_wait(nxt)
                        mkidx(nxt)
                        gather_issue(nxt)

                plsc.subcore_barrier()
                pltpu.sync_copy(agg_sh.at[pl.ds(s * stripe, stripe)],
                                agg_ref.at[c, k, pl.ds(s * stripe, stripe)])
                plsc.subcore_barrier()

        direction(hs_t_ref, code_tc_ref, ss_t_sh, sd_c_sh,
                  p_tc_ref, agg_c_ref, nt, nc, rc, sh_tc)
        direction(hs_c_ref, code_ct_ref, ss_c_sh, sd_t_sh,
                  p_ct_ref, agg_t_ref, nc, nt, rt, sh_ct)

    return sc_fn


def _head_body(a0, a1, z0, z1, qg, wo, bo, w1, b1, w2, b2, out):
    agg = (a0[...] + a1[...]) / (z0[...] + z1[...] + 1e-16)
    f = jnp.dot(jnp.maximum(agg, 0.0), wo[...], preferred_element_type=F32) + bo[...]
    h = jnp.maximum(jnp.dot(f, w1[...], preferred_element_type=F32) + qg[...] + b1[...], 0.0)
    out[...] = jnp.dot(h, w2[...], preferred_element_type=F32) + b2[...]


def _head(a0, a1, z0, z1, qg, wo, bo, w1, b1, w2, b2, bs):
    n = qg.shape[0]
    row = lambda i: (i, 0)
    fix = lambda i: (0, 0)
    out = pl.pallas_call(
        _head_body,
        grid=(n // bs,),
        in_specs=[
            pl.BlockSpec((bs, D), row), pl.BlockSpec((bs, D), row),
            pl.BlockSpec((bs, 1), row), pl.BlockSpec((bs, 1), row),
            pl.BlockSpec((bs, D), row),
            pl.BlockSpec((D, D), fix), pl.BlockSpec((1, D), fix),
            pl.BlockSpec((D, D), fix), pl.BlockSpec((1, D), fix),
            pl.BlockSpec((D, 1), fix), pl.BlockSpec((1, 1), fix),
        ],
        out_specs=pl.BlockSpec((bs, 1), row),
        out_shape=jax.ShapeDtypeStruct((n, 1), F32),
    )(a0, a1, z0, z1, qg, wo, bo.reshape(1, D), w1, b1.reshape(1, D),
      w2, b2.reshape(1, 1))
    return out[:, 0]


def kernel(x_table, x_column, edge_index_tc, edge_index_ct, batch_table,
           batch_column, queries, W_src_tc, W_dst_tc, a_src_tc, a_dst_tc,
           W_src_ct, W_dst_ct, a_src_ct, a_dst_ct, W_out_t, b_out_t,
           W_out_c, b_out_c, W_t1, b_t1, W_t2, b_t2, W_c1, b_c1, W_c2, b_c2):
    nt = x_table.shape[0]
    nc = x_column.shape[0]
    e = edge_index_tc.shape[1]
    b = queries.shape[0]

    hs_tc, ss_tc, sd_ct = _dense_pre(x_table, W_src_tc, a_src_tc,
                                     W_dst_ct, a_dst_ct, bs=2000)
    hs_ct, ss_ct, sd_tc = _dense_pre(x_column, W_src_ct, a_src_ct,
                                     W_dst_tc, a_dst_tc, bs=2000)
    qw_t, qw_c = _dense_qw(queries, W_t1[D:], W_c1[D:])

    def _slab_stack(m):
        return jnp.concatenate([m[:, k * SW:(k + 1) * SW] for k in range(NSLAB)], 0)

    hs_t_all = _slab_stack(hs_tc)
    hs_c_all = _slab_stack(hs_ct)
    qw_t_all = _slab_stack(qw_t)
    qw_c_all = _slab_stack(qw_c)

    def _pack_scores(v, n):
        npad = ((n + SW - 1) // SW) * SW
        return jnp.concatenate([v[:, 0], jnp.zeros((npad - n,), F32)]).reshape(-1, SW)

    ss_tc_p = _pack_scores(ss_tc, nt)
    sd_tc_p = _pack_scores(sd_tc, nc)
    ss_ct_p = _pack_scores(ss_ct, nc)
    sd_ct_p = _pack_scores(sd_ct, nt)
    nst = ss_tc_p.shape[0]
    nsc = sd_tc_p.shape[0]

    quantum = NCORE * NSUB * 2 * EB
    epad = ((e + quantum - 1) // quantum) * quantum
    pad = epad - e

    def _pack_edges(ei, n_src, n_dst):
        shift = (n_src - 1).bit_length()
        src = jnp.concatenate([ei[0].astype(I32), jnp.zeros((pad,), I32)])
        dst = jnp.concatenate([ei[1].astype(I32),
                               jnp.full((pad,), n_dst, I32)])
        return src | (dst << shift)

    code_tc = _pack_edges(edge_index_tc, nt, nc)
    code_ct = _pack_edges(edge_index_ct, nc, nt)

    rc = ((nc + EB) // EB) * EB
    rt = ((nt + EB) // EB) * EB
    qidx_c = jnp.concatenate(
        [batch_column.astype(I32), jnp.zeros((rc - nc,), I32)])
    qidx_t = jnp.concatenate(
        [batch_table.astype(I32), jnp.zeros((rt - nt,), I32)])

    sc_fn = _sc_all(nt, nc, epad, rc, rt, nst, nsc, b)
    agg_c5, agg_t5, qg_c4, qg_t4, _p1, _p2 = sc_fn(
        hs_t_all, hs_c_all, ss_tc_p, sd_tc_p, ss_ct_p, sd_ct_p,
        code_tc, code_ct,
        qw_c_all, qidx_c, qw_t_all, qidx_t)

    def _unslab(a4, n):
        return jnp.concatenate([a4[k, :n] for k in range(NSLAB)], axis=1)

    agg_c0 = _unslab(agg_c5[0, :NSLAB], nc)
    agg_c1 = _unslab(agg_c5[1, :NSLAB], nc)
    z_c0 = agg_c5[0, NSLAB, :nc, 0:1]
    z_c1 = agg_c5[1, NSLAB, :nc, 0:1]
    agg_t0 = _unslab(agg_t5[0, :NSLAB], nt)
    agg_t1 = _unslab(agg_t5[1, :NSLAB], nt)
    z_t0 = agg_t5[0, NSLAB, :nt, 0:1]
    z_t1 = agg_t5[1, NSLAB, :nt, 0:1]
    qg_c = _unslab(qg_c4, nc)
    qg_t = _unslab(qg_t4, nt)

    table_probs = _head(agg_t0, agg_t1, z_t0, z_t1, qg_t,
                        W_out_t, b_out_t, W_t1[:D], b_t1, W_t2, b_t2, bs=2000)
    column_probs = _head(agg_c0, agg_c1, z_c0, z_c1, qg_c,
                         W_out_c, b_out_c, W_c1[:D], b_c1, W_c2, b_c2, bs=2000)

    return (table_probs, column_probs)


# speedup vs baseline: 8.2874x; 1.0098x over previous
import functools

import jax
import jax.numpy as jnp
from jax import lax
from jax.experimental import pallas as pl
from jax.experimental.pallas import tpu as pltpu
from jax.experimental.pallas import tpu_sc as plsc

F32 = jnp.float32
I32 = jnp.int32

D = 128
SW = 32
NSLAB = D // SW
NCORE = 2
NSUB = 16
NLANE = 16
EB = 128


def _pre_body(x_ref, ws_ref, as_ref, wd_ref, ad_ref, hs_ref, ss_ref, sd_ref):
    x = x_ref[...]
    hs = jnp.dot(x, ws_ref[...], preferred_element_type=F32)
    hs_ref[...] = hs
    ss_ref[...] = jnp.dot(hs, as_ref[...], preferred_element_type=F32)
    hd = jnp.dot(x, wd_ref[...], preferred_element_type=F32)
    sd_ref[...] = jnp.dot(hd, ad_ref[...], preferred_element_type=F32)


def _dense_pre(x, w_src, a_src, w_dst, a_dst, bs):
    n = x.shape[0]
    return pl.pallas_call(
        _pre_body,
        grid=(n // bs,),
        in_specs=[
            pl.BlockSpec((bs, D), lambda i: (i, 0)),
            pl.BlockSpec((D, D), lambda i: (0, 0)),
            pl.BlockSpec((D, 1), lambda i: (0, 0)),
            pl.BlockSpec((D, D), lambda i: (0, 0)),
            pl.BlockSpec((D, 1), lambda i: (0, 0)),
        ],
        out_specs=[
            pl.BlockSpec((bs, D), lambda i: (i, 0)),
            pl.BlockSpec((bs, 1), lambda i: (i, 0)),
            pl.BlockSpec((bs, 1), lambda i: (i, 0)),
        ],
        out_shape=[
            jax.ShapeDtypeStruct((n, D), F32),
            jax.ShapeDtypeStruct((n, 1), F32),
            jax.ShapeDtypeStruct((n, 1), F32),
        ],
    )(x, w_src, a_src.reshape(D, 1), w_dst, a_dst.reshape(D, 1))


def _qw_body(q_ref, wt_ref, wc_ref, qt_ref, qc_ref):
    q = q_ref[...]
    qt_ref[...] = jnp.dot(q, wt_ref[...], preferred_element_type=F32)
    qc_ref[...] = jnp.dot(q, wc_ref[...], preferred_element_type=F32)


def _dense_qw(queries, w_t1q, w_c1q):
    b = queries.shape[0]
    return pl.pallas_call(
        _qw_body,
        grid=(1,),
        in_specs=[
            pl.BlockSpec((b, D), lambda i: (0, 0)),
            pl.BlockSpec((D, D), lambda i: (0, 0)),
            pl.BlockSpec((D, D), lambda i: (0, 0)),
        ],
        out_specs=[
            pl.BlockSpec((b, D), lambda i: (0, 0)),
            pl.BlockSpec((b, D), lambda i: (0, 0)),
        ],
        out_shape=[
            jax.ShapeDtypeStruct((b, D), F32),
            jax.ShapeDtypeStruct((b, D), F32),
        ],
    )(queries, w_t1q, w_c1q)


def _sc_all(nt, nc, epad, rc, rt, nst, nsc, b):
    ept = epad // (NSUB * NCORE)
    DB = 2 * EB
    nb2 = ept // DB
    ngrp = EB // NLANE
    sh_tc = (nt - 1).bit_length()
    sh_ct = (nc - 1).bit_length()

    mesh = plsc.VectorSubcoreMesh(
        core_axis_name="c", subcore_axis_name="s",
        num_cores=NCORE, num_subcores=NSUB)

    @functools.partial(
        pl.kernel,
        out_type=(
            jax.ShapeDtypeStruct((NCORE, NSLAB + 1, rc, SW), F32),
            jax.ShapeDtypeStruct((NCORE, NSLAB + 1, rt, SW), F32),
            jax.ShapeDtypeStruct((NSLAB, rc, SW), F32),
            jax.ShapeDtypeStruct((NSLAB, rt, SW), F32),
            jax.ShapeDtypeStruct((epad,), F32),
            jax.ShapeDtypeStruct((epad,), F32),
        ),
        mesh=mesh,
        compiler_params=pltpu.CompilerParams(
            needs_layout_passes=False, use_tc_tiling_on_sc=False),
        scratch_types=[
            pltpu.VMEM((EB, SW), F32),
            pltpu.VMEM((EB, SW), F32),
            pltpu.VMEM((EB, SW), F32),
            pltpu.VMEM((EB, SW), F32),
            pltpu.VMEM((64, SW), F32),
            pltpu.VMEM((DB,), I32),
            pltpu.VMEM((DB,), F32),
            pltpu.VMEM((DB,), I32),
            pltpu.VMEM((DB,), F32),
            pltpu.VMEM((EB,), I32),
            pltpu.VMEM((EB,), I32),
            pltpu.VMEM((EB,), I32),
            pltpu.VMEM((EB,), I32),
            pltpu.VMEM((EB,), I32),
            pltpu.VMEM((EB,), I32),
            pltpu.VMEM((EB,), I32),
            pltpu.VMEM((EB,), I32),
            pltpu.VMEM((EB,), I32),
            pltpu.VMEM_SHARED((nst, SW), F32),
            pltpu.VMEM_SHARED((nsc, SW), F32),
            pltpu.VMEM_SHARED((nsc, SW), F32),
            pltpu.VMEM_SHARED((nst, SW), F32),
            pltpu.VMEM_SHARED((rc, SW), F32),
            pltpu.SemaphoreType.DMA,
            pltpu.SemaphoreType.DMA,
        ],
    )
    def sc_fn(hs_t_ref, hs_c_ref, ss_tc_ref, sd_tc_ref, ss_ct_ref, sd_ct_ref,
              code_tc_ref, code_ct_ref,
              qw_c_ref, qidx_c_ref, qw_t_ref, qidx_t_ref,
              agg_c_ref, agg_t_ref, qg_c_ref, qg_t_ref, p_tc_ref, p_ct_ref,
              rows_a, rows_b, rss, rsd, zbuf, codeb, p_b, codeb2, p_b2,
              ridx_a, ridx_b, ridx_a2, ridx_b2,
              dloc0, dloc1, dloc2, dloc3, qidxb,
              ss_t_sh, sd_c_sh, ss_c_sh, sd_t_sh, agg_sh, sem, sem_l):
        c = lax.axis_index("c")
        s = lax.axis_index("s")
        wid = c * NSUB + s
        iot = jnp.arange(NLANE, dtype=I32)

        @pl.when(s == 0)
        def _stage():
            pltpu.sync_copy(ss_tc_ref, ss_t_sh)
            pltpu.sync_copy(sd_tc_ref, sd_c_sh)
            pltpu.sync_copy(ss_ct_ref, ss_c_sh)
            pltpu.sync_copy(sd_ct_ref, sd_t_sh)

        @pl.loop(0, 64)
        def _zb(i):
            for j in range(SW // NLANE):
                zbuf[i, pl.ds(j * NLANE, NLANE)] = jnp.zeros((NLANE,), F32)

        def q_gather(qidx_ref, qw_ref, qg_ref, nqb, nq):
            @pl.loop(wid, nqb, step=NCORE * NSUB)
            def _qg(blk):
                pltpu.sync_copy(qidx_ref.at[pl.ds(blk * EB, EB)], qidxb)
                for k in range(NSLAB):
                    @pl.loop(0, ngrp)
                    def _adj(jj):
                        off = jj * NLANE
                        qidx = qidxb[pl.ds(off, NLANE)]
                        ridx_a[pl.ds(off, NLANE)] = qidx + k * nq
                    pltpu.async_copy(qw_ref.at[ridx_a], rows_a, sem).wait()
                    pltpu.sync_copy(rows_a,
                                    qg_ref.at[k, pl.ds(blk * EB, EB)])

        q_gather(qidx_c_ref, qw_c_ref, qg_c_ref, rc // EB, b)
        q_gather(qidx_t_ref, qw_t_ref, qg_t_ref, rt // EB, b)

        plsc.subcore_barrier()

        def clear_acc(span):
            nfull = span // 64
            rem = span % 64
            @pl.loop(0, nfull)
            def _zz(i):
                pltpu.sync_copy(zbuf, agg_sh.at[pl.ds(s * span + i * 64, 64)])
            if rem:
                pltpu.sync_copy(zbuf.at[pl.ds(0, rem)],
                                agg_sh.at[pl.ds(s * span + nfull * 64, rem)])
            plsc.subcore_barrier()

        def direction(hs_ref, code_ref, ss_sh, sd_sh, p_ref,
                      agg_ref, n_src, n_dst, rd, shift):
            ebase = wid * ept
            stripe = rd // NSUB
            mask = (1 << shift) - 1
            dlocs = (dloc0, dloc1)

            Z0 = (codeb, p_b, ridx_a, ridx_b, dloc0, rss, rsd)
            Z1 = (codeb2, p_b2, ridx_a2, ridx_b2, dloc2, rows_a, rows_b)
            nbz = ept // EB

            def zload_issue(ib, S):
                pltpu.async_copy(code_ref.at[pl.ds(ebase + ib * EB, EB)],
                                 S[0].at[pl.ds(0, EB)], sem_l)

            def zload_wait(S):
                pltpu.make_async_copy(code_ref.at[pl.ds(ebase, EB)],
                                      S[0].at[pl.ds(0, EB)], sem_l).wait()

            def zmkidx(S):
                cb, _, ra, rb, dl, _, _ = S

                @pl.loop(0, ngrp)
                def _mk(jj):
                    off = jj * NLANE
                    cd = cb[pl.ds(off, NLANE)]
                    si = jnp.minimum(cd & mask, n_src - 1)
                    di = jnp.minimum(cd >> shift, n_dst)
                    dg = jnp.minimum(di, n_dst - 1)
                    ra[pl.ds(off, NLANE)] = si >> 5
                    rb[pl.ds(off, NLANE)] = dg >> 5
                    dl[pl.ds(off, NLANE)] = di

            def zgather_issue(S):
                pltpu.async_copy(ss_sh.at[S[2]], S[5], sem)
                pltpu.async_copy(sd_sh.at[S[3]], S[6], sem)

            def zgather_wait(S):
                pltpu.make_async_copy(ss_sh.at[S[2]], S[5], sem).wait()
                pltpu.make_async_copy(sd_sh.at[S[3]], S[6], sem).wait()

            def zscore_scatter(ib, S):
                cb, pb, _, _, dl, r_ss, r_sd = S

                @pl.loop(0, ngrp)
                def _score(jj):
                    off = jj * NLANE
                    ioff = iot + off
                    cd = cb[pl.ds(off, NLANE)]
                    si = jnp.minimum(cd & mask, n_src - 1)
                    dg = jnp.minimum(cd >> shift, n_dst - 1)
                    sv = plsc.load_gather(r_ss, [ioff, si & 31])
                    dv = plsc.load_gather(r_sd, [ioff, dg & 31])
                    al = sv + dv
                    al = jnp.where(al >= 0.0, al, al * 0.2)
                    p = jnp.exp(al)
                    pb[pl.ds(off, NLANE)] = p
                    plsc.store_scatter(
                        r_ss, [ioff, jnp.zeros((NLANE,), I32)], p)

                pltpu.sync_copy(pb.at[pl.ds(0, EB)],
                                p_ref.at[pl.ds(ebase + ib * EB, EB)])
                pltpu.sync_copy(r_ss, agg_sh.at[dl], add=True)

            clear_acc(stripe)

            pltpu.sync_copy(code_ref.at[pl.ds(ebase, EB)],
                            codeb.at[pl.ds(0, EB)])
            zmkidx(Z0)
            zgather_issue(Z0)

            npz = (nbz - 1) // 2

            @pl.loop(0, npz)
            def _zpair(i2):
                for par, (cur, nxt) in ((0, (Z0, Z1)), (1, (Z1, Z0))):
                    ib = i2 * 2 + par
                    zload_issue(ib + 1, nxt)
                    zgather_wait(cur)
                    zscore_scatter(ib, cur)
                    zload_wait(nxt)
                    zmkidx(nxt)
                    zgather_issue(nxt)

            for ib in range(2 * npz, nbz):
                cur, nxt = (Z0, Z1) if ib % 2 == 0 else (Z1, Z0)
                if ib + 1 < nbz:
                    zload_issue(ib + 1, nxt)
                zgather_wait(cur)
                zscore_scatter(ib, cur)
                if ib + 1 < nbz:
                    zload_wait(nxt)
                    zmkidx(nxt)
                    zgather_issue(nxt)

            plsc.subcore_barrier()
            pltpu.sync_copy(agg_sh.at[pl.ds(s * stripe, stripe)],
                            agg_ref.at[c, NSLAB, pl.ds(s * stripe, stripe)])
            plsc.subcore_barrier()

            B0 = (codeb, p_b, ridx_a, ridx_b, dloc0, dloc1, rows_a, rows_b)
            B1 = (codeb2, p_b2, ridx_a2, ridx_b2, dloc2, dloc3, rss, rsd)

            @pl.loop(0, NSLAB)
            def _slab(k):
                clear_acc(stripe)

                def load_issue(ib, S):
                    eb = ebase + ib * DB
                    pltpu.async_copy(code_ref.at[pl.ds(eb, DB)], S[0], sem_l)
                    pltpu.async_copy(p_ref.at[pl.ds(eb, DB)], S[1], sem_l)

                def load_wait(S):
                    pltpu.make_async_copy(
                        code_ref.at[pl.ds(ebase, DB)], S[0], sem_l).wait()
                    pltpu.make_async_copy(
                        p_ref.at[pl.ds(ebase, DB)], S[1], sem_l).wait()

                def mkidx(S):
                    cb, _, ra, rb, d0, d1, _, _ = S

                    @pl.loop(0, ngrp)
                    def _mk(jj):
                        off = jj * NLANE
                        cd0 = cb[pl.ds(off, NLANE)]
                        cd1 = cb[pl.ds(EB + off, NLANE)]
                        ra[pl.ds(off, NLANE)] = (
                            jnp.minimum(cd0 & mask, n_src - 1) + k * n_src)
                        rb[pl.ds(off, NLANE)] = (
                            jnp.minimum(cd1 & mask, n_src - 1) + k * n_src)
                        d0[pl.ds(off, NLANE)] = jnp.minimum(cd0 >> shift, n_dst)
                        d1[pl.ds(off, NLANE)] = jnp.minimum(cd1 >> shift, n_dst)

                def gather_issue(S):
                    pltpu.async_copy(hs_ref.at[S[2]], S[6], sem)
                    pltpu.async_copy(hs_ref.at[S[3]], S[7], sem)

                def gather_wait(S):
                    pltpu.make_async_copy(hs_ref.at[S[2]], S[6], sem).wait()
                    pltpu.make_async_copy(hs_ref.at[S[3]], S[7], sem).wait()

                def scale_scatter(S):
                    _, pb, _, _, d0, d1, ra_, rb_ = S
                    for h, rows_h in ((0, ra_), (1, rb_)):
                        @pl.loop(0, ngrp)
                        def _scale(g):
                            row0 = g * NLANE
                            for ii in range(NLANE):
                                pv = plsc.load_gather(
                                    pb, [jnp.full((NLANE,),
                                                  h * EB + row0 + ii, I32)])
                                r = row0 + ii
                                for j in range(SW // NLANE):
                                    rows_h[r, pl.ds(j * NLANE, NLANE)] = (
                                        rows_h[r, pl.ds(j * NLANE, NLANE)] * pv)
                    pltpu.sync_copy(ra_, agg_sh.at[d0], add=True)
                    pltpu.sync_copy(rb_, agg_sh.at[d1], add=True)

                pltpu.sync_copy(code_ref.at[pl.ds(ebase, DB)], codeb)
                pltpu.sync_copy(p_ref.at[pl.ds(ebase, DB)], p_b)
                mkidx(B0)
                gather_issue(B0)

                npairs = (nb2 - 1) // 2

                @pl.loop(0, npairs)
                def _pair(i2):
                    for par, (cur, nxt) in ((0, (B0, B1)), (1, (B1, B0))):
                        ib = i2 * 2 + par
                        load_issue(ib + 1, nxt)
                        gather_wait(cur)
                        scale_scatter(cur)
                        load_wait(nxt)
                        mkidx(nxt)
                        gather_issue(nxt)

                for ib in range(2 * npairs, nb2):
                    cur, nxt = (B0, B1) if ib % 2 == 0 else (B1, B0)
                    if ib + 1 < nb2:
                        load_issue(ib + 1, nxt)
                    gather_wait(cur)
                    scale_scatter(cur)
                    if ib + 1 < nb2:
                        load_wait(nxt)
                        mkidx(nxt)
                        gather_issue(nxt)

                plsc.subcore_barrier()
                pltpu.sync_copy(agg_sh.at[pl.ds(s * stripe, stripe)],
                                agg_ref.at[c, k, pl.ds(s * stripe, stripe)])
                plsc.subcore_barrier()

        direction(hs_t_ref, code_tc_ref, ss_t_sh, sd_c_sh,
                  p_tc_ref, agg_c_ref, nt, nc, rc, sh_tc)
        direction(hs_c_ref, code_ct_ref, ss_c_sh, sd_t_sh,
                  p_ct_ref, agg_t_ref, nc, nt, rt, sh_ct)

    return sc_fn


def _head_body(a0, a1, z0, z1, qg, wo, bo, w1, b1, w2, b2, out):
    agg = (a0[...] + a1[...]) / (z0[...] + z1[...] + 1e-16)
    f = jnp.dot(jnp.maximum(agg, 0.0), wo[...], preferred_element_type=F32) + bo[...]
    h = jnp.maximum(jnp.dot(f, w1[...], preferred_element_type=F32) + qg[...] + b1[...], 0.0)
    out[...] = jnp.dot(h, w2[...], preferred_element_type=F32) + b2[...]


def _head(a0, a1, z0, z1, qg, wo, bo, w1, b1, w2, b2, bs):
    n = qg.shape[0]
    row = lambda i: (i, 0)
    fix = lambda i: (0, 0)
    out = pl.pallas_call(
        _head_body,
        grid=(n // bs,),
        in_specs=[
            pl.BlockSpec((bs, D), row), pl.BlockSpec((bs, D), row),
            pl.BlockSpec((bs, 1), row), pl.BlockSpec((bs, 1), row),
            pl.BlockSpec((bs, D), row),
            pl.BlockSpec((D, D), fix), pl.BlockSpec((1, D), fix),
            pl.BlockSpec((D, D), fix), pl.BlockSpec((1, D), fix),
            pl.BlockSpec((D, 1), fix), pl.BlockSpec((1, 1), fix),
        ],
        out_specs=pl.BlockSpec((bs, 1), row),
        out_shape=jax.ShapeDtypeStruct((n, 1), F32),
    )(a0, a1, z0, z1, qg, wo, bo.reshape(1, D), w1, b1.reshape(1, D),
      w2, b2.reshape(1, 1))
    return out[:, 0]


def kernel(x_table, x_column, edge_index_tc, edge_index_ct, batch_table,
           batch_column, queries, W_src_tc, W_dst_tc, a_src_tc, a_dst_tc,
           W_src_ct, W_dst_ct, a_src_ct, a_dst_ct, W_out_t, b_out_t,
           W_out_c, b_out_c, W_t1, b_t1, W_t2, b_t2, W_c1, b_c1, W_c2, b_c2):
    nt = x_table.shape[0]
    nc = x_column.shape[0]
    e = edge_index_tc.shape[1]
    b = queries.shape[0]

    hs_tc, ss_tc, sd_ct = _dense_pre(x_table, W_src_tc, a_src_tc,
                                     W_dst_ct, a_dst_ct, bs=2000)
    hs_ct, ss_ct, sd_tc = _dense_pre(x_column, W_src_ct, a_src_ct,
                                     W_dst_tc, a_dst_tc, bs=2000)
    qw_t, qw_c = _dense_qw(queries, W_t1[D:], W_c1[D:])

    def _slab_stack(m):
        return jnp.concatenate([m[:, k * SW:(k + 1) * SW] for k in range(NSLAB)], 0)

    hs_t_all = _slab_stack(hs_tc)
    hs_c_all = _slab_stack(hs_ct)
    qw_t_all = _slab_stack(qw_t)
    qw_c_all = _slab_stack(qw_c)

    def _pack_scores(v, n):
        npad = ((n + SW - 1) // SW) * SW
        return jnp.concatenate([v[:, 0], jnp.zeros((npad - n,), F32)]).reshape(-1, SW)

    ss_tc_p = _pack_scores(ss_tc, nt)
    sd_tc_p = _pack_scores(sd_tc, nc)
    ss_ct_p = _pack_scores(ss_ct, nc)
    sd_ct_p = _pack_scores(sd_ct, nt)
    nst = ss_tc_p.shape[0]
    nsc = sd_tc_p.shape[0]

    quantum = NCORE * NSUB * 2 * EB
    epad = ((e + quantum - 1) // quantum) * quantum
    pad = epad - e

    def _pack_edges(ei, n_src, n_dst):
        shift = (n_src - 1).bit_length()
        src = jnp.concatenate([ei[0].astype(I32), jnp.zeros((pad,), I32)])
        dst = jnp.concatenate([ei[1].astype(I32),
                               jnp.full((pad,), n_dst, I32)])
        return src | (dst << shift)

    code_tc = _pack_edges(edge_index_tc, nt, nc)
    code_ct = _pack_edges(edge_index_ct, nc, nt)

    rc = ((nc + EB) // EB) * EB
    rt = ((nt + EB) // EB) * EB
    qidx_c = jnp.concatenate(
        [batch_column.astype(I32), jnp.zeros((rc - nc,), I32)])
    qidx_t = jnp.concatenate(
        [batch_table.astype(I32), jnp.zeros((rt - nt,), I32)])

    sc_fn = _sc_all(nt, nc, epad, rc, rt, nst, nsc, b)
    agg_c5, agg_t5, qg_c4, qg_t4, _p1, _p2 = sc_fn(
        hs_t_all, hs_c_all, ss_tc_p, sd_tc_p, ss_ct_p, sd_ct_p,
        code_tc, code_ct,
        qw_c_all, qidx_c, qw_t_all, qidx_t)

    def _unslab(a4, n):
        return jnp.concatenate([a4[k, :n] for k in range(NSLAB)], axis=1)

    agg_c0 = _unslab(agg_c5[0, :NSLAB], nc)
    agg_c1 = _unslab(agg_c5[1, :NSLAB], nc)
    z_c0 = agg_c5[0, NSLAB, :nc, 0:1]
    z_c1 = agg_c5[1, NSLAB, :nc, 0:1]
    agg_t0 = _unslab(agg_t5[0, :NSLAB], nt)
    agg_t1 = _unslab(agg_t5[1, :NSLAB], nt)
    z_t0 = agg_t5[0, NSLAB, :nt, 0:1]
    z_t1 = agg_t5[1, NSLAB, :nt, 0:1]
    qg_c = _unslab(qg_c4, nc)
    qg_t = _unslab(qg_t4, nt)

    table_probs = _head(agg_t0, agg_t1, z_t0, z_t1, qg_t,
                        W_out_t, b_out_t, W_t1[:D], b_t1, W_t2, b_t2, bs=2000)
    column_probs = _head(agg_c0, agg_c1, z_c0, z_c1, qg_c,
                         W_out_c, b_out_c, W_c1[:D], b_c1, W_c2, b_c2, bs=2000)

    return (table_probs, column_probs)


# speedup vs baseline: 9.0252x; 1.0890x over previous
import functools

import jax
import jax.numpy as jnp
from jax import lax
from jax.experimental import pallas as pl
from jax.experimental.pallas import tpu as pltpu
from jax.experimental.pallas import tpu_sc as plsc

F32 = jnp.float32
I32 = jnp.int32

D = 128
SW = 32
NSLAB = D // SW
NCORE = 2
NSUB = 16
NLANE = 16
EB = 128


def _pre_body(x_ref, ws_ref, as_ref, wd_ref, ad_ref, hs_ref, ss_ref, sd_ref):
    x = x_ref[...]
    hs = jnp.dot(x, ws_ref[...], preferred_element_type=F32)
    hs_ref[...] = hs
    ss_ref[...] = jnp.dot(hs, as_ref[...], preferred_element_type=F32)
    hd = jnp.dot(x, wd_ref[...], preferred_element_type=F32)
    sd_ref[...] = jnp.dot(hd, ad_ref[...], preferred_element_type=F32)


def _dense_pre(x, w_src, a_src, w_dst, a_dst, bs):
    n = x.shape[0]
    return pl.pallas_call(
        _pre_body,
        grid=(n // bs,),
        in_specs=[
            pl.BlockSpec((bs, D), lambda i: (i, 0)),
            pl.BlockSpec((D, D), lambda i: (0, 0)),
            pl.BlockSpec((D, 1), lambda i: (0, 0)),
            pl.BlockSpec((D, D), lambda i: (0, 0)),
            pl.BlockSpec((D, 1), lambda i: (0, 0)),
        ],
        out_specs=[
            pl.BlockSpec((bs, D), lambda i: (i, 0)),
            pl.BlockSpec((bs, 1), lambda i: (i, 0)),
            pl.BlockSpec((bs, 1), lambda i: (i, 0)),
        ],
        out_shape=[
            jax.ShapeDtypeStruct((n, D), F32),
            jax.ShapeDtypeStruct((n, 1), F32),
            jax.ShapeDtypeStruct((n, 1), F32),
        ],
    )(x, w_src, a_src.reshape(D, 1), w_dst, a_dst.reshape(D, 1))


def _qw_body(q_ref, wt_ref, wc_ref, qt_ref, qc_ref):
    q = q_ref[...]
    qt_ref[...] = jnp.dot(q, wt_ref[...], preferred_element_type=F32)
    qc_ref[...] = jnp.dot(q, wc_ref[...], preferred_element_type=F32)


def _dense_qw(queries, w_t1q, w_c1q):
    b = queries.shape[0]
    return pl.pallas_call(
        _qw_body,
        grid=(1,),
        in_specs=[
            pl.BlockSpec((b, D), lambda i: (0, 0)),
            pl.BlockSpec((D, D), lambda i: (0, 0)),
            pl.BlockSpec((D, D), lambda i: (0, 0)),
        ],
        out_specs=[
            pl.BlockSpec((b, D), lambda i: (0, 0)),
            pl.BlockSpec((b, D), lambda i: (0, 0)),
        ],
        out_shape=[
            jax.ShapeDtypeStruct((b, D), F32),
            jax.ShapeDtypeStruct((b, D), F32),
        ],
    )(queries, w_t1q, w_c1q)


def _sc_all(nt, nc, epad, rc, rt, nst, nsc, b):
    ept = epad // (NSUB * NCORE)
    DB = 2 * EB
    nb2 = ept // DB
    ngrp = EB // NLANE
    sh_tc = (nt - 1).bit_length()
    sh_ct = (nc - 1).bit_length()

    mesh = plsc.VectorSubcoreMesh(
        core_axis_name="c", subcore_axis_name="s",
        num_cores=NCORE, num_subcores=NSUB)

    @functools.partial(
        pl.kernel,
        out_type=(
            jax.ShapeDtypeStruct((NCORE, NSLAB + 1, rc, SW), F32),
            jax.ShapeDtypeStruct((NCORE, NSLAB + 1, rt, SW), F32),
            jax.ShapeDtypeStruct((NSLAB, rc, SW), F32),
            jax.ShapeDtypeStruct((NSLAB, rt, SW), F32),
            jax.ShapeDtypeStruct((epad,), F32),
            jax.ShapeDtypeStruct((epad,), F32),
        ),
        mesh=mesh,
        compiler_params=pltpu.CompilerParams(
            needs_layout_passes=False, use_tc_tiling_on_sc=False),
        scratch_types=[
            pltpu.VMEM((EB, SW), F32),
            pltpu.VMEM((EB, SW), F32),
            pltpu.VMEM((EB, SW), F32),
            pltpu.VMEM((EB, SW), F32),
            pltpu.VMEM((64, SW), F32),
            pltpu.VMEM((DB,), I32),
            pltpu.VMEM((DB,), F32),
            pltpu.VMEM((DB,), I32),
            pltpu.VMEM((DB,), F32),
            pltpu.VMEM((EB,), I32),
            pltpu.VMEM((EB,), I32),
            pltpu.VMEM((EB,), I32),
            pltpu.VMEM((EB,), I32),
            pltpu.VMEM((EB,), I32),
            pltpu.VMEM((EB,), I32),
            pltpu.VMEM((EB,), I32),
            pltpu.VMEM((EB,), I32),
            pltpu.VMEM((EB,), I32),
            pltpu.VMEM_SHARED((nst, SW), F32),
            pltpu.VMEM_SHARED((nsc, SW), F32),
            pltpu.VMEM_SHARED((nsc, SW), F32),
            pltpu.VMEM_SHARED((nst, SW), F32),
            pltpu.VMEM_SHARED((rc, SW), F32),
            pltpu.SemaphoreType.DMA,
            pltpu.SemaphoreType.DMA,
        ],
    )
    def sc_fn(hs_t_ref, hs_c_ref, ss_tc_ref, sd_tc_ref, ss_ct_ref, sd_ct_ref,
              code_tc_ref, code_ct_ref,
              qw_c_ref, qidx_c_ref, qw_t_ref, qidx_t_ref,
              agg_c_ref, agg_t_ref, qg_c_ref, qg_t_ref, p_tc_ref, p_ct_ref,
              rows_a, rows_b, rss, rsd, zbuf, codeb, p_b, codeb2, p_b2,
              ridx_a, ridx_b, ridx_a2, ridx_b2,
              dloc0, dloc1, dloc2, dloc3, qidxb,
              ss_t_sh, sd_c_sh, ss_c_sh, sd_t_sh, agg_sh, sem, sem_l):
        c = lax.axis_index("c")
        s = lax.axis_index("s")
        wid = c * NSUB + s
        iot = jnp.arange(NLANE, dtype=I32)

        @pl.when(s == 0)
        def _stage():
            pltpu.sync_copy(ss_tc_ref, ss_t_sh)
            pltpu.sync_copy(sd_tc_ref, sd_c_sh)
            pltpu.sync_copy(ss_ct_ref, ss_c_sh)
            pltpu.sync_copy(sd_ct_ref, sd_t_sh)

        @pl.loop(0, 64)
        def _zb(i):
            for j in range(SW // NLANE):
                zbuf[i, pl.ds(j * NLANE, NLANE)] = jnp.zeros((NLANE,), F32)

        def q_gather(qidx_ref, qw_ref, qg_ref, nqb, nq):
            @pl.loop(wid, nqb, step=NCORE * NSUB)
            def _qg(blk):
                pltpu.sync_copy(qidx_ref.at[pl.ds(blk * EB, EB)], qidxb)
                for k in range(NSLAB):
                    @pl.loop(0, ngrp)
                    def _adj(jj):
                        off = jj * NLANE
                        qidx = qidxb[pl.ds(off, NLANE)]
                        ridx_a[pl.ds(off, NLANE)] = qidx + k * nq
                    pltpu.async_copy(qw_ref.at[ridx_a], rows_a, sem).wait()
                    pltpu.sync_copy(rows_a,
                                    qg_ref.at[k, pl.ds(blk * EB, EB)])

        q_gather(qidx_c_ref, qw_c_ref, qg_c_ref, rc // EB, b)
        q_gather(qidx_t_ref, qw_t_ref, qg_t_ref, rt // EB, b)

        plsc.subcore_barrier()

        def clear_acc(span):
            nfull = span // 64
            rem = span % 64
            @pl.loop(0, nfull)
            def _zz(i):
                pltpu.sync_copy(zbuf, agg_sh.at[pl.ds(s * span + i * 64, 64)])
            if rem:
                pltpu.sync_copy(zbuf.at[pl.ds(0, rem)],
                                agg_sh.at[pl.ds(s * span + nfull * 64, rem)])
            plsc.subcore_barrier()

        def direction(hs_ref, code_ref, ss_sh, sd_sh, p_ref,
                      agg_ref, n_src, n_dst, rd, shift):
            ebase = wid * ept
            stripe = rd // NSUB
            mask = (1 << shift) - 1
            dlocs = (dloc0, dloc1)

            Z0 = (codeb, p_b, ridx_a, ridx_b, dloc0, rss, rsd)
            Z1 = (codeb2, p_b2, ridx_a2, ridx_b2, dloc2, rows_a, rows_b)
            nbz = ept // EB

            def zload_issue(ib, S):
                pltpu.async_copy(code_ref.at[pl.ds(ebase + ib * EB, EB)],
                                 S[0].at[pl.ds(0, EB)], sem_l)

            def zload_wait(S):
                pltpu.make_async_copy(code_ref.at[pl.ds(ebase, EB)],
                                      S[0].at[pl.ds(0, EB)], sem_l).wait()

            def zmkidx(S):
                cb, _, ra, rb, dl, _, _ = S

                @pl.loop(0, ngrp)
                def _mk(jj):
                    off = jj * NLANE
                    cd = cb[pl.ds(off, NLANE)]
                    si = jnp.minimum(cd & mask, n_src - 1)
                    di = jnp.minimum(cd >> shift, n_dst)
                    dg = jnp.minimum(di, n_dst - 1)
                    ra[pl.ds(off, NLANE)] = si >> 5
                    rb[pl.ds(off, NLANE)] = dg >> 5
                    dl[pl.ds(off, NLANE)] = di

            def zgather_issue(S):
                pltpu.async_copy(ss_sh.at[S[2]], S[5], sem)
                pltpu.async_copy(sd_sh.at[S[3]], S[6], sem)

            def zgather_wait(S):
                pltpu.make_async_copy(ss_sh.at[S[2]], S[5], sem).wait()
                pltpu.make_async_copy(sd_sh.at[S[3]], S[6], sem).wait()

            def zscore_scatter(ib, S):
                cb, pb, _, _, dl, r_ss, r_sd = S

                @pl.loop(0, ngrp)
                def _score(jj):
                    off = jj * NLANE
                    ioff = iot + off
                    cd = cb[pl.ds(off, NLANE)]
                    si = jnp.minimum(cd & mask, n_src - 1)
                    dg = jnp.minimum(cd >> shift, n_dst - 1)
                    sv = plsc.load_gather(r_ss, [ioff, si & 31])
                    dv = plsc.load_gather(r_sd, [ioff, dg & 31])
                    al = sv + dv
                    al = jnp.where(al >= 0.0, al, al * 0.2)
                    p = jnp.exp(al)
                    pb[pl.ds(off, NLANE)] = p
                    plsc.store_scatter(
                        r_ss, [ioff, jnp.zeros((NLANE,), I32)], p)

                pltpu.sync_copy(pb.at[pl.ds(0, EB)],
                                p_ref.at[pl.ds(ebase + ib * EB, EB)])
                pltpu.sync_copy(r_ss, agg_sh.at[dl], add=True)

            clear_acc(stripe)

            pltpu.sync_copy(code_ref.at[pl.ds(ebase, EB)],
                            codeb.at[pl.ds(0, EB)])
            zmkidx(Z0)
            zgather_issue(Z0)

            npz = (nbz - 1) // 2

            @pl.loop(0, npz)
            def _zpair(i2):
                for par, (cur, nxt) in ((0, (Z0, Z1)), (1, (Z1, Z0))):
                    ib = i2 * 2 + par
                    zload_issue(ib + 1, nxt)
                    zgather_wait(cur)
                    zscore_scatter(ib, cur)
                    zload_wait(nxt)
                    zmkidx(nxt)
                    zgather_issue(nxt)

            for ib in range(2 * npz, nbz):
                cur, nxt = (Z0, Z1) if ib % 2 == 0 else (Z1, Z0)
                if ib + 1 < nbz:
                    zload_issue(ib + 1, nxt)
                zgather_wait(cur)
                zscore_scatter(ib, cur)
                if ib + 1 < nbz:
                    zload_wait(nxt)
                    zmkidx(nxt)
                    zgather_issue(nxt)

            plsc.subcore_barrier()
            pltpu.sync_copy(agg_sh.at[pl.ds(s * stripe, stripe)],
                            agg_ref.at[c, NSLAB, pl.ds(s * stripe, stripe)])
            plsc.subcore_barrier()

            B0 = (codeb, p_b, ridx_a, ridx_b, dloc0, dloc1, rows_a, rows_b)
            B1 = (codeb2, p_b2, ridx_a2, ridx_b2, dloc2, dloc3, rss, rsd)

            @pl.loop(0, NSLAB)
            def _slab(k):
                clear_acc(stripe)

                def load_issue(ib, S):
                    eb = ebase + ib * DB
                    pltpu.async_copy(code_ref.at[pl.ds(eb, DB)], S[0], sem_l)
                    pltpu.async_copy(p_ref.at[pl.ds(eb, DB)], S[1], sem_l)

                def load_wait(S):
                    pltpu.make_async_copy(
                        code_ref.at[pl.ds(ebase, DB)], S[0], sem_l).wait()
                    pltpu.make_async_copy(
                        p_ref.at[pl.ds(ebase, DB)], S[1], sem_l).wait()

                def mkidx(S):
                    cb, _, ra, rb, d0, d1, _, _ = S

                    @pl.loop(0, ngrp)
                    def _mk(jj):
                        off = jj * NLANE
                        cd0 = cb[pl.ds(off, NLANE)]
                        cd1 = cb[pl.ds(EB + off, NLANE)]
                        ra[pl.ds(off, NLANE)] = (
                            jnp.minimum(cd0 & mask, n_src - 1) + k * n_src)
                        rb[pl.ds(off, NLANE)] = (
                            jnp.minimum(cd1 & mask, n_src - 1) + k * n_src)
                        d0[pl.ds(off, NLANE)] = jnp.minimum(cd0 >> shift, n_dst)
                        d1[pl.ds(off, NLANE)] = jnp.minimum(cd1 >> shift, n_dst)

                def gather_issue(S):
                    pltpu.async_copy(hs_ref.at[S[2]], S[6], sem)
                    pltpu.async_copy(hs_ref.at[S[3]], S[7], sem)

                def gather_wait(S):
                    pltpu.make_async_copy(hs_ref.at[S[2]], S[6], sem).wait()
                    pltpu.make_async_copy(hs_ref.at[S[3]], S[7], sem).wait()

                def scale_scatter(S):
                    _, pb, _, _, d0, d1, ra_, rb_ = S
                    for h, rows_h in ((0, ra_), (1, rb_)):
                        @pl.loop(0, ngrp)
                        def _scale(g):
                            row0 = g * NLANE
                            for ii in range(NLANE):
                                pv = plsc.load_gather(
                                    pb, [jnp.full((NLANE,),
                                                  h * EB + row0 + ii, I32)])
                                r = row0 + ii
                                for j in range(SW // NLANE):
                                    rows_h[r, pl.ds(j * NLANE, NLANE)] = (
                                        rows_h[r, pl.ds(j * NLANE, NLANE)] * pv)
                    pltpu.sync_copy(ra_, agg_sh.at[d0], add=True)
                    pltpu.sync_copy(rb_, agg_sh.at[d1], add=True)

                pltpu.sync_copy(code_ref.at[pl.ds(ebase, DB)], codeb)
                pltpu.sync_copy(p_ref.at[pl.ds(ebase, DB)], p_b)
                mkidx(B0)
                gather_issue(B0)

                npairs = (nb2 - 1) // 2

                @pl.loop(0, npairs)
                def _pair(i2):
                    for par, (cur, nxt) in ((0, (B0, B1)), (1, (B1, B0))):
                        ib = i2 * 2 + par
                        load_issue(ib + 1, nxt)
                        gather_wait(cur)
                        scale_scatter(cur)
                        load_wait(nxt)
                        mkidx(nxt)
                        gather_issue(nxt)

                for ib in range(2 * npairs, nb2):
                    cur, nxt = (B0, B1) if ib % 2 == 0 else (B1, B0)
                    if ib + 1 < nb2:
                        load_issue(ib + 1, nxt)
                    gather_wait(cur)
                    scale_scatter(cur)
                    if ib + 1 < nb2:
                        load_wait(nxt)
                        mkidx(nxt)
                        gather_issue(nxt)

                plsc.subcore_barrier()
                pltpu.sync_copy(agg_sh.at[pl.ds(s * stripe, stripe)],
                                agg_ref.at[c, k, pl.ds(s * stripe, stripe)])
                plsc.subcore_barrier()

        direction(hs_t_ref, code_tc_ref, ss_t_sh, sd_c_sh,
                  p_tc_ref, agg_c_ref, nt, nc, rc, sh_tc)
        direction(hs_c_ref, code_ct_ref, ss_c_sh, sd_t_sh,
                  p_ct_ref, agg_t_ref, nc, nt, rt, sh_ct)

    return sc_fn


def _head_body(*refs):
    (a00, a01, a02, a03, a10, a11, a12, a13, z0, z1,
     q0, q1, q2, q3, wo, bo, w1, b1, w2, b2, out) = refs
    a0 = jnp.concatenate([a00[...], a01[...], a02[...], a03[...]], axis=1)
    a1 = jnp.concatenate([a10[...], a11[...], a12[...], a13[...]], axis=1)
    qg = jnp.concatenate([q0[...], q1[...], q2[...], q3[...]], axis=1)
    agg = (a0 + a1) / (z0[...] + z1[...] + 1e-16)
    f = jnp.dot(jnp.maximum(agg, 0.0), wo[...], preferred_element_type=F32) + bo[...]
    h = jnp.maximum(jnp.dot(f, w1[...], preferred_element_type=F32) + qg + b1[...], 0.0)
    out[...] = jnp.dot(h, w2[...], preferred_element_type=F32) + b2[...]


def _head(agg5, z0, z1, qg4, n, wo, bo, w1, b1, w2, b2, bs):
    row = lambda i: (i, 0)
    fix = lambda i: (0, 0)
    slab = pl.BlockSpec((bs, SW), row)
    out = pl.pallas_call(
        _head_body,
        grid=(n // bs,),
        in_specs=[
            slab, slab, slab, slab, slab, slab, slab, slab,
            pl.BlockSpec((bs, 1), row), pl.BlockSpec((bs, 1), row),
            slab, slab, slab, slab,
            pl.BlockSpec((D, D), fix), pl.BlockSpec((1, D), fix),
            pl.BlockSpec((D, D), fix), pl.BlockSpec((1, D), fix),
            pl.BlockSpec((D, 1), fix), pl.BlockSpec((1, 1), fix),
        ],
        out_specs=pl.BlockSpec((bs, 1), row),
        out_shape=jax.ShapeDtypeStruct((n, 1), F32),
    )(*[agg5[0, k, :n] for k in range(NSLAB)],
      *[agg5[1, k, :n] for k in range(NSLAB)],
      z0, z1,
      *[qg4[k, :n] for k in range(NSLAB)],
      wo, bo.reshape(1, D), w1, b1.reshape(1, D), w2, b2.reshape(1, 1))
    return out[:, 0]


def kernel(x_table, x_column, edge_index_tc, edge_index_ct, batch_table,
           batch_column, queries, W_src_tc, W_dst_tc, a_src_tc, a_dst_tc,
           W_src_ct, W_dst_ct, a_src_ct, a_dst_ct, W_out_t, b_out_t,
           W_out_c, b_out_c, W_t1, b_t1, W_t2, b_t2, W_c1, b_c1, W_c2, b_c2):
    nt = x_table.shape[0]
    nc = x_column.shape[0]
    e = edge_index_tc.shape[1]
    b = queries.shape[0]

    hs_tc, ss_tc, sd_ct = _dense_pre(x_table, W_src_tc, a_src_tc,
                                     W_dst_ct, a_dst_ct, bs=2000)
    hs_ct, ss_ct, sd_tc = _dense_pre(x_column, W_src_ct, a_src_ct,
                                     W_dst_tc, a_dst_tc, bs=2000)
    qw_t, qw_c = _dense_qw(queries, W_t1[D:], W_c1[D:])

    def _slab_stack(m):
        return jnp.concatenate([m[:, k * SW:(k + 1) * SW] for k in range(NSLAB)], 0)

    hs_t_all = _slab_stack(hs_tc)
    hs_c_all = _slab_stack(hs_ct)
    qw_t_all = _slab_stack(qw_t)
    qw_c_all = _slab_stack(qw_c)

    def _pack_scores(v, n):
        npad = ((n + SW - 1) // SW) * SW
        return jnp.concatenate([v[:, 0], jnp.zeros((npad - n,), F32)]).reshape(-1, SW)

    ss_tc_p = _pack_scores(ss_tc, nt)
    sd_tc_p = _pack_scores(sd_tc, nc)
    ss_ct_p = _pack_scores(ss_ct, nc)
    sd_ct_p = _pack_scores(sd_ct, nt)
    nst = ss_tc_p.shape[0]
    nsc = sd_tc_p.shape[0]

    quantum = NCORE * NSUB * 2 * EB
    epad = ((e + quantum - 1) // quantum) * quantum
    pad = epad - e

    def _pack_edges(ei, n_src, n_dst):
        shift = (n_src - 1).bit_length()
        src = jnp.concatenate([ei[0].astype(I32), jnp.zeros((pad,), I32)])
        dst = jnp.concatenate([ei[1].astype(I32),
                               jnp.full((pad,), n_dst, I32)])
        return src | (dst << shift)

    code_tc = _pack_edges(edge_index_tc, nt, nc)
    code_ct = _pack_edges(edge_index_ct, nc, nt)

    rc = ((nc + EB) // EB) * EB
    rt = ((nt + EB) // EB) * EB
    qidx_c = jnp.concatenate(
        [batch_column.astype(I32), jnp.zeros((rc - nc,), I32)])
    qidx_t = jnp.concatenate(
        [batch_table.astype(I32), jnp.zeros((rt - nt,), I32)])

    sc_fn = _sc_all(nt, nc, epad, rc, rt, nst, nsc, b)
    agg_c5, agg_t5, qg_c4, qg_t4, _p1, _p2 = sc_fn(
        hs_t_all, hs_c_all, ss_tc_p, sd_tc_p, ss_ct_p, sd_ct_p,
        code_tc, code_ct,
        qw_c_all, qidx_c, qw_t_all, qidx_t)

    z_c0 = agg_c5[0, NSLAB, :nc, 0:1]
    z_c1 = agg_c5[1, NSLAB, :nc, 0:1]
    z_t0 = agg_t5[0, NSLAB, :nt, 0:1]
    z_t1 = agg_t5[1, NSLAB, :nt, 0:1]

    table_probs = _head(agg_t5, z_t0, z_t1, qg_t4, nt,
                        W_out_t, b_out_t, W_t1[:D], b_t1, W_t2, b_t2, bs=2000)
    column_probs = _head(agg_c5, z_c0, z_c1, qg_c4, nc,
                         W_out_c, b_out_c, W_c1[:D], b_c1, W_c2, b_c2, bs=2000)

    return (table_probs, column_probs)
